# Initial kernel scaffold; baseline (speedup 1.0000x reference)
import functools

import jax
import jax.numpy as jnp
from jax import lax
from jax.experimental import pallas as pl
from jax.experimental.pallas import tpu as pltpu
from jax.experimental.pallas import tpu_sc as plsc

N = 10000
E = 320000
D = 128
F = 64
G = 25
GP = 32
NI = 3
NG = 64
MAXZ = 100
RC = 6.0
LOG2 = 0.6931471805599453

NC = 2
NS = 16
NW = NC * NS

CHUNK = 128
CPW = 79
E_PAD = NW * CPW * CHUNK
NBLK = 79
N_PAD = NBLK * 128
RPS = N_PAD // NS
ZR = RPS // 4
EB = 512
EBLK = E_PAD // EB


def _ssp(x):
    return jax.nn.softplus(x) - LOG2


def _node_init_body(z_ref, emb_ref, w_ref, x_ref, v_ref):
    z = z_ref[...][:, 0]
    oh = (z[:, None] == lax.broadcasted_iota(jnp.int32, (128, 128), 1))
    x = jnp.dot(oh.astype(jnp.float32), emb_ref[...],
                preferred_element_type=jnp.float32)
    x_ref[...] = x
    v_ref[...] = jnp.dot(x, w_ref[...], preferred_element_type=jnp.float32)


def _node_init(z2, emb_pad, w0):
    return pl.pallas_call(
        _node_init_body,
        grid=(NBLK,),
        in_specs=[
            pl.BlockSpec((128, 1), lambda n: (n, 0)),
            pl.BlockSpec((128, D), lambda n: (0, 0)),
            pl.BlockSpec((D, F), lambda n: (0, 0)),
        ],
        out_specs=[
            pl.BlockSpec((128, D), lambda n: (n, 0)),
            pl.BlockSpec((128, F), lambda n: (n, 0)),
        ],
        out_shape=[
            jax.ShapeDtypeStruct((N_PAD, D), jnp.float32),
            jax.ShapeDtypeStruct((N_PAD, F), jnp.float32),
        ],
    )(z2, emb_pad, w0)


def _filters_body(dr_ref, em_ref, w1_ref, b1_ref, w2_ref, b2_ref,
                  wf0_ref, wf1_ref, wf2_ref):
    dr = dr_ref[...]
    offs = lax.broadcasted_iota(jnp.float32, (1, GP), 1) * (RC / (G - 1))
    coeff = -0.5 / (RC / (G - 1)) ** 2
    dexp = jnp.exp(coeff * (dr - offs) ** 2)
    cut = 0.5 * (jnp.cos(jnp.pi / RC * dr) + 1.0)
    cut = cut * (dr < RC).astype(jnp.float32) * em_ref[...]
    h = _ssp(jnp.dot(dexp, w1_ref[...], preferred_element_type=jnp.float32)
             + b1_ref[...])
    wf = (jnp.dot(h, w2_ref[...], preferred_element_type=jnp.float32)
          + b2_ref[...]) * cut
    wf0_ref[...] = wf[:, 0:F]
    wf1_ref[...] = wf[:, F:2 * F]
    wf2_ref[...] = wf[:, 2 * F:3 * F]


def _filters(dr2, em2, w1c, b1c, w2bd, b2c):
    return pl.pallas_call(
        _filters_body,
        grid=(EBLK,),
        in_specs=[
            pl.BlockSpec((EB, 1), lambda e: (e, 0)),
            pl.BlockSpec((EB, 1), lambda e: (e, 0)),
            pl.BlockSpec((GP, NI * F), lambda e: (0, 0)),
            pl.BlockSpec((1, NI * F), lambda e: (0, 0)),
            pl.BlockSpec((NI * F, NI * F), lambda e: (0, 0)),
            pl.BlockSpec((1, NI * F), lambda e: (0, 0)),
        ],
        out_specs=[pl.BlockSpec((EB, F), lambda e: (e, 0))] * NI,
        out_shape=[jax.ShapeDtypeStruct((E_PAD, F), jnp.float32)] * NI,
    )(dr2, em2, w1c, b1c, w2bd, b2c)


def _sc_conv_body(v_hbm, wf_hbm, s_hbm, r_hbm, out_hbm,
                  sidx, ridx, rows, wfv, zbuf, agg_sh, sem):
    c = lax.axis_index("c")
    s = lax.axis_index("s")
    w = c * NS + s

    pltpu.async_copy(s_hbm.at[pl.ds(w * CPW, CPW)], sidx, sem).wait()
    pltpu.async_copy(r_hbm.at[pl.ds(w * CPW, CPW)], ridx, sem).wait()

    zero16 = jnp.zeros((16,), jnp.float32)

    @pl.loop(0, ZR)
    def _(r):
        for cc in range(F // 16):
            zbuf.at[r, pl.ds(cc * 16, 16)][...] = zero16

    for t in range(RPS // ZR):
        pltpu.async_copy(zbuf, agg_sh.at[pl.ds(s * RPS + t * ZR, ZR)],
                         sem).wait()
    plsc.subcore_barrier()

    @pl.loop(0, CPW)
    def _(j):
        base = (w * CPW + j) * CHUNK
        pltpu.async_copy(wf_hbm.at[pl.ds(base, CHUNK)], wfv, sem).wait()
        pltpu.async_copy(v_hbm.at[sidx.at[j]], rows, sem).wait()

        @pl.loop(0, CHUNK)
        def _(r):
            for cc in range(F // 16):
                sl = pl.ds(cc * 16, 16)
                rows.at[r, sl][...] = rows.at[r, sl][...] * wfv.at[r, sl][...]

        pltpu.async_copy(rows, agg_sh.at[ridx.at[j]], sem, add=True).wait()

    plsc.subcore_barrier()
    pltpu.async_copy(agg_sh.at[pl.ds(s * RPS, RPS)],
                     out_hbm.at[c, pl.ds(s * RPS, RPS)], sem).wait()


_SC_MESH = plsc.VectorSubcoreMesh(core_axis_name="c", subcore_axis_name="s",
                                  num_cores=NC, num_subcores=NS)

_sc_conv = pl.kernel(
    _sc_conv_body,
    out_type=jax.ShapeDtypeStruct((NC, N_PAD, F), jnp.float32),
    mesh=_SC_MESH,
    scratch_types=[
        pltpu.VMEM((CPW, CHUNK), jnp.int32),
        pltpu.VMEM((CPW, CHUNK), jnp.int32),
        pltpu.VMEM((CHUNK, F), jnp.float32),
        pltpu.VMEM((CHUNK, F), jnp.float32),
        pltpu.VMEM((ZR, F), jnp.float32),
        pltpu.VMEM_SHARED((N_PAD, F), jnp.float32),
        pltpu.SemaphoreType.DMA,
    ],
)


def _interact_body(agg_ref, x_ref, fw_ref, fb_ref, dw_ref, db_ref, nw_ref,
                   xo_ref, vo_ref):
    agg = agg_ref[0] + agg_ref[1]
    t = _ssp(jnp.dot(agg, fw_ref[...], preferred_element_type=jnp.float32)
             + fb_ref[...])
    t = jnp.dot(t, dw_ref[...], preferred_element_type=jnp.float32) + db_ref[...]
    xn = x_ref[...] + t
    xo_ref[...] = xn
    vo_ref[...] = jnp.dot(xn, nw_ref[...], preferred_element_type=jnp.float32)


def _interact(agg2, x, fw, fb, dw, db, nw):
    return pl.pallas_call(
        _interact_body,
        grid=(NBLK,),
        in_specs=[
            pl.BlockSpec((NC, 128, F), lambda n: (0, n, 0)),
            pl.BlockSpec((128, D), lambda n: (n, 0)),
            pl.BlockSpec((F, D), lambda n: (0, 0)),
            pl.BlockSpec((1, D), lambda n: (0, 0)),
            pl.BlockSpec((D, D), lambda n: (0, 0)),
            pl.BlockSpec((1, D), lambda n: (0, 0)),
            pl.BlockSpec((D, F), lambda n: (0, 0)),
        ],
        out_specs=[
            pl.BlockSpec((128, D), lambda n: (n, 0)),
            pl.BlockSpec((128, F), lambda n: (n, 0)),
        ],
        out_shape=[
            jax.ShapeDtypeStruct((N_PAD, D), jnp.float32),
            jax.ShapeDtypeStruct((N_PAD, F), jnp.float32),
        ],
    )(agg2, x, fw, fb, dw, db, nw)


def _pool_body(x_ref, w1_ref, b1_ref, w2_ref, b2_ref, nm_ref, gi_ref, o_ref):
    h = _ssp(jnp.dot(x_ref[...], w1_ref[...],
                     preferred_element_type=jnp.float32) + b1_ref[...])
    yi = jnp.sum(h * w2_ref[...], axis=1) + b2_ref[0, 0]
    yi = yi * nm_ref[...][:, 0]
    gi = gi_ref[...][:, 0]
    oh = (gi[:, None] == lax.broadcasted_iota(jnp.int32, (128, NG), 1))
    part = jnp.sum(yi[:, None] * oh.astype(jnp.float32), axis=0)

    @pl.when(pl.program_id(0) == 0)
    def _():
        o_ref[...] = jnp.zeros_like(o_ref)

    o_ref[...] += part[None, :]


def _pool(x, w1, b1, w2r, b2, nm2, gi2):
    return pl.pallas_call(
        _pool_body,
        grid=(NBLK,),
        in_specs=[
            pl.BlockSpec((128, D), lambda n: (n, 0)),
            pl.BlockSpec((D, 32), lambda n: (0, 0)),
            pl.BlockSpec((1, 32), lambda n: (0, 0)),
            pl.BlockSpec((1, 32), lambda n: (0, 0)),
            pl.BlockSpec((1, 1), lambda n: (0, 0)),
            pl.BlockSpec((128, 1), lambda n: (n, 0)),
            pl.BlockSpec((128, 1), lambda n: (n, 0)),
        ],
        out_specs=pl.BlockSpec((1, NG), lambda n: (0, 0)),
        out_shape=jax.ShapeDtypeStruct((1, NG), jnp.float32),
    )(x, w1, b1, w2r, b2, nm2, gi2)


def kernel(z, dR, senders, receivers, graph_idx, node_mask, edge_mask,
           embed, in2f_W, fnet_W1, fnet_b1, fnet_W2, fnet_b2,
           f2out_W, f2out_b, dense_W, dense_b,
           aw_W1, aw_b1, aw_W2, aw_b2):
    f32 = jnp.float32
    z2 = jnp.pad(z.astype(jnp.int32), (0, N_PAD - N)).reshape(N_PAD, 1)
    dr2 = jnp.pad(dR, (0, E_PAD - E), constant_values=2.0 * RC) \
             .reshape(E_PAD, 1)
    em2 = jnp.pad(edge_mask, (0, E_PAD - E)).reshape(E_PAD, 1)
    s2 = jnp.pad(senders.astype(jnp.int32), (0, E_PAD - E)) \
            .reshape(NW * CPW, CHUNK)
    r2 = jnp.pad(receivers.astype(jnp.int32), (0, E_PAD - E)) \
            .reshape(NW * CPW, CHUNK)
    nm2 = jnp.pad(node_mask, (0, N_PAD - N)).reshape(N_PAD, 1)
    gi2 = jnp.pad(graph_idx.astype(jnp.int32), (0, N_PAD - N)) \
             .reshape(N_PAD, 1)

    emb_pad = jnp.pad(embed, ((0, 128 - MAXZ), (0, 0)))
    w1c = jnp.pad(jnp.concatenate([fnet_W1[i] for i in range(NI)], axis=1),
                  ((0, GP - G), (0, 0)))
    b1c = jnp.concatenate([fnet_b1[i] for i in range(NI)]).reshape(1, NI * F)
    w2bd = jnp.zeros((NI * F, NI * F), f32)
    for i in range(NI):
        w2bd = w2bd.at[i * F:(i + 1) * F, i * F:(i + 1) * F].set(fnet_W2[i])
    b2c = jnp.concatenate([fnet_b2[i] for i in range(NI)]).reshape(1, NI * F)

    x, v = _node_init(z2, emb_pad, in2f_W[0])
    wfs = _filters(dr2, em2, w1c, b1c, w2bd, b2c)
    for i in range(NI):
        agg2 = _sc_conv(v, wfs[i], s2, r2)
        nxt = in2f_W[i + 1] if i + 1 < NI else in2f_W[0]
        x, v = _interact(agg2, x, f2out_W[i], f2out_b[i].reshape(1, D),
                         dense_W[i], dense_b[i].reshape(1, D), nxt)

    out = _pool(x, aw_W1, aw_b1.reshape(1, 32), aw_W2.reshape(1, 32),
                aw_b2.reshape(1, 1), nm2, gi2)
    return out.reshape(NG)


# speedup vs baseline: 1.7217x; 1.7217x over previous
import functools

import jax
import jax.numpy as jnp
from jax import lax
from jax.experimental import pallas as pl
from jax.experimental.pallas import tpu as pltpu
from jax.experimental.pallas import tpu_sc as plsc

N = 10000
E = 320000
D = 128
F = 64
G = 25
GP = 32
NI = 3
NG = 64
MAXZ = 100
RC = 6.0
LOG2 = 0.6931471805599453

NC = 2
NS = 16
NW = NC * NS

CHUNK = 128
CPW = 80
E_PAD = NW * CPW * CHUNK
NBLK = 79
N_PAD = NBLK * 128
RPS = N_PAD // NS
ZR = RPS
EB = 512
EBLK = E_PAD // EB


def _ssp(x):
    return jax.nn.softplus(x) - LOG2


def _node_init_body(z_ref, emb_ref, w_ref, x_ref, v_ref):
    z = z_ref[...][:, 0]
    oh = (z[:, None] == lax.broadcasted_iota(jnp.int32, (128, 128), 1))
    x = jnp.dot(oh.astype(jnp.float32), emb_ref[...],
                preferred_element_type=jnp.float32)
    x_ref[...] = x
    v_ref[...] = jnp.dot(x, w_ref[...], preferred_element_type=jnp.float32)


def _node_init(z2, emb_pad, w0):
    return pl.pallas_call(
        _node_init_body,
        grid=(NBLK,),
        in_specs=[
            pl.BlockSpec((128, 1), lambda n: (n, 0)),
            pl.BlockSpec((128, D), lambda n: (0, 0)),
            pl.BlockSpec((D, F), lambda n: (0, 0)),
        ],
        out_specs=[
            pl.BlockSpec((128, D), lambda n: (n, 0)),
            pl.BlockSpec((128, F), lambda n: (n, 0)),
        ],
        out_shape=[
            jax.ShapeDtypeStruct((N_PAD, D), jnp.float32),
            jax.ShapeDtypeStruct((N_PAD, F), jnp.float32),
        ],
    )(z2, emb_pad, w0)


def _filters_body(dr_ref, em_ref, w1_ref, b1_ref, w2_ref, b2_ref,
                  wf0_ref, wf1_ref, wf2_ref):
    dr = dr_ref[...]
    offs = lax.broadcasted_iota(jnp.int32, (1, GP), 1).astype(jnp.float32) \
        * (RC / (G - 1))
    coeff = -0.5 / (RC / (G - 1)) ** 2
    dexp = jnp.exp(coeff * (dr - offs) ** 2)
    cut = 0.5 * (jnp.cos(jnp.pi / RC * dr) + 1.0)
    cut = cut * (dr < RC).astype(jnp.float32) * em_ref[...]
    h = _ssp(jnp.dot(dexp, w1_ref[...], preferred_element_type=jnp.float32)
             + b1_ref[...])
    wf = (jnp.dot(h, w2_ref[...], preferred_element_type=jnp.float32)
          + b2_ref[...]) * cut
    wf0_ref[...] = wf[:, 0:F]
    wf1_ref[...] = wf[:, F:2 * F]
    wf2_ref[...] = wf[:, 2 * F:3 * F]


def _filters(dr2, em2, w1c, b1c, w2bd, b2c):
    return pl.pallas_call(
        _filters_body,
        grid=(EBLK,),
        in_specs=[
            pl.BlockSpec((EB, 1), lambda e: (e, 0)),
            pl.BlockSpec((EB, 1), lambda e: (e, 0)),
            pl.BlockSpec((GP, NI * F), lambda e: (0, 0)),
            pl.BlockSpec((1, NI * F), lambda e: (0, 0)),
            pl.BlockSpec((NI * F, NI * F), lambda e: (0, 0)),
            pl.BlockSpec((1, NI * F), lambda e: (0, 0)),
        ],
        out_specs=[pl.BlockSpec((EB, F), lambda e: (e, 0))] * NI,
        out_shape=[jax.ShapeDtypeStruct((E_PAD, F), jnp.float32)] * NI,
    )(dr2, em2, w1c, b1c, w2bd, b2c)


def _sc_conv_body(v_hbm, wf_hbm, s_hbm, r_hbm, out_hbm,
                  sidx, ridx, rows, wfv, zbuf, agg_sh, sem):
    c = lax.axis_index("c")
    s = lax.axis_index("s")
    w = c * NS + s

    pltpu.async_copy(s_hbm.at[pl.ds(w * CPW, CPW)], sidx, sem).wait()
    pltpu.async_copy(r_hbm.at[pl.ds(w * CPW, CPW)], ridx, sem).wait()

    zero16 = jnp.zeros((16,), jnp.float32)

    @pl.loop(0, ZR)
    def _(r):
        for cc in range(F // 16):
            zbuf.at[r, pl.ds(cc * 16, 16)][...] = zero16

    pltpu.async_copy(zbuf, agg_sh.at[pl.ds(s * RPS, RPS)], sem).wait()
    plsc.subcore_barrier()

    @pl.loop(0, CPW)
    def _(j):
        base = (w * CPW + j) * CHUNK
        pltpu.async_copy(wf_hbm.at[pl.ds(base, CHUNK)], wfv, sem).wait()
        pltpu.async_copy(v_hbm.at[sidx.at[j]], rows, sem).wait()

        @pl.loop(0, CHUNK)
        def _(r):
            for cc in range(F // 16):
                sl = pl.ds(cc * 16, 16)
                rows.at[r, sl][...] = rows.at[r, sl][...] * wfv.at[r, sl][...]

        pltpu.async_copy(rows, agg_sh.at[ridx.at[j]], sem, add=True).wait()

    plsc.subcore_barrier()
    pltpu.async_copy(agg_sh.at[pl.ds(s * RPS, RPS)],
                     out_hbm.at[c, pl.ds(s * RPS, RPS)], sem).wait()


@functools.cache
def _sc_conv_kernel():
    mesh = plsc.VectorSubcoreMesh(core_axis_name="c", subcore_axis_name="s",
                                  num_cores=NC, num_subcores=NS)
    return pl.kernel(
        _sc_conv_body,
        out_type=jax.ShapeDtypeStruct((NC, N_PAD, F), jnp.float32),
        mesh=mesh,
        compiler_params=pltpu.CompilerParams(use_tc_tiling_on_sc=False),
        scratch_types=[
            pltpu.VMEM((CPW, CHUNK), jnp.int32),
            pltpu.VMEM((CPW, CHUNK), jnp.int32),
            pltpu.VMEM((CHUNK, F), jnp.float32),
            pltpu.VMEM((CHUNK, F), jnp.float32),
            pltpu.VMEM((ZR, F), jnp.float32),
            pltpu.VMEM_SHARED((N_PAD, F), jnp.float32),
            pltpu.SemaphoreType.DMA,
        ],
    )


def _sc_conv(v, wf, s2, r2):
    return _sc_conv_kernel()(v, wf, s2, r2)


def _interact_body(agg_ref, x_ref, fw_ref, fb_ref, dw_ref, db_ref, nw_ref,
                   xo_ref, vo_ref):
    agg = agg_ref[0] + agg_ref[1]
    t = _ssp(jnp.dot(agg, fw_ref[...], preferred_element_type=jnp.float32)
             + fb_ref[...])
    t = jnp.dot(t, dw_ref[...], preferred_element_type=jnp.float32) + db_ref[...]
    xn = x_ref[...] + t
    xo_ref[...] = xn
    vo_ref[...] = jnp.dot(xn, nw_ref[...], preferred_element_type=jnp.float32)


def _interact(agg2, x, fw, fb, dw, db, nw):
    return pl.pallas_call(
        _interact_body,
        grid=(NBLK,),
        in_specs=[
            pl.BlockSpec((NC, 128, F), lambda n: (0, n, 0)),
            pl.BlockSpec((128, D), lambda n: (n, 0)),
            pl.BlockSpec((F, D), lambda n: (0, 0)),
            pl.BlockSpec((1, D), lambda n: (0, 0)),
            pl.BlockSpec((D, D), lambda n: (0, 0)),
            pl.BlockSpec((1, D), lambda n: (0, 0)),
            pl.BlockSpec((D, F), lambda n: (0, 0)),
        ],
        out_specs=[
            pl.BlockSpec((128, D), lambda n: (n, 0)),
            pl.BlockSpec((128, F), lambda n: (n, 0)),
        ],
        out_shape=[
            jax.ShapeDtypeStruct((N_PAD, D), jnp.float32),
            jax.ShapeDtypeStruct((N_PAD, F), jnp.float32),
        ],
    )(agg2, x, fw, fb, dw, db, nw)


def _pool_body(x_ref, w1_ref, b1_ref, w2_ref, b2_ref, nm_ref, gi_ref, o_ref):
    h = _ssp(jnp.dot(x_ref[...], w1_ref[...],
                     preferred_element_type=jnp.float32) + b1_ref[...])
    yi = jnp.sum(h * w2_ref[...], axis=1) + b2_ref[0, 0]
    yi = yi * nm_ref[...][:, 0]
    gi = gi_ref[...][:, 0]
    oh = (gi[:, None] == lax.broadcasted_iota(jnp.int32, (128, NG), 1))
    part = jnp.sum(yi[:, None] * oh.astype(jnp.float32), axis=0)

    @pl.when(pl.program_id(0) == 0)
    def _():
        o_ref[...] = jnp.zeros_like(o_ref)

    o_ref[...] += part[None, :]


def _pool(x, w1, b1, w2r, b2, nm2, gi2):
    return pl.pallas_call(
        _pool_body,
        grid=(NBLK,),
        in_specs=[
            pl.BlockSpec((128, D), lambda n: (n, 0)),
            pl.BlockSpec((D, 32), lambda n: (0, 0)),
            pl.BlockSpec((1, 32), lambda n: (0, 0)),
            pl.BlockSpec((1, 32), lambda n: (0, 0)),
            pl.BlockSpec((1, 1), lambda n: (0, 0)),
            pl.BlockSpec((128, 1), lambda n: (n, 0)),
            pl.BlockSpec((128, 1), lambda n: (n, 0)),
        ],
        out_specs=pl.BlockSpec((1, NG), lambda n: (0, 0)),
        out_shape=jax.ShapeDtypeStruct((1, NG), jnp.float32),
    )(x, w1, b1, w2r, b2, nm2, gi2)


def kernel(z, dR, senders, receivers, graph_idx, node_mask, edge_mask,
           embed, in2f_W, fnet_W1, fnet_b1, fnet_W2, fnet_b2,
           f2out_W, f2out_b, dense_W, dense_b,
           aw_W1, aw_b1, aw_W2, aw_b2):
    f32 = jnp.float32
    z2 = jnp.pad(z.astype(jnp.int32), (0, N_PAD - N)).reshape(N_PAD, 1)
    dr2 = jnp.pad(dR, (0, E_PAD - E), constant_values=2.0 * RC) \
             .reshape(E_PAD, 1)
    em2 = jnp.pad(edge_mask, (0, E_PAD - E)).reshape(E_PAD, 1)
    s2 = jnp.pad(senders.astype(jnp.int32), (0, E_PAD - E)) \
            .reshape(NW * CPW, CHUNK)
    r2 = jnp.pad(receivers.astype(jnp.int32), (0, E_PAD - E)) \
            .reshape(NW * CPW, CHUNK)
    nm2 = jnp.pad(node_mask, (0, N_PAD - N)).reshape(N_PAD, 1)
    gi2 = jnp.pad(graph_idx.astype(jnp.int32), (0, N_PAD - N)) \
             .reshape(N_PAD, 1)

    emb_pad = jnp.pad(embed, ((0, 128 - MAXZ), (0, 0)))
    w1c = jnp.pad(jnp.concatenate([fnet_W1[i] for i in range(NI)], axis=1),
                  ((0, GP - G), (0, 0)))
    b1c = jnp.concatenate([fnet_b1[i] for i in range(NI)]).reshape(1, NI * F)
    w2bd = jnp.zeros((NI * F, NI * F), f32)
    for i in range(NI):
        w2bd = w2bd.at[i * F:(i + 1) * F, i * F:(i + 1) * F].set(fnet_W2[i])
    b2c = jnp.concatenate([fnet_b2[i] for i in range(NI)]).reshape(1, NI * F)

    x, v = _node_init(z2, emb_pad, in2f_W[0])
    wfs = _filters(dr2, em2, w1c, b1c, w2bd, b2c)
    for i in range(NI):
        agg2 = _sc_conv(v, wfs[i], s2, r2)
        nxt = in2f_W[i + 1] if i + 1 < NI else in2f_W[0]
        x, v = _interact(agg2, x, f2out_W[i], f2out_b[i].reshape(1, D),
                         dense_W[i], dense_b[i].reshape(1, D), nxt)

    out = _pool(x, aw_W1, aw_b1.reshape(1, 32), aw_W2.reshape(1, 32),
                aw_b2.reshape(1, 1), nm2, gi2)
    return out.reshape(NG)


# speedup vs baseline: 1.9766x; 1.1481x over previous
import functools

import jax
import jax.numpy as jnp
from jax import lax
from jax.experimental import pallas as pl
from jax.experimental.pallas import tpu as pltpu
from jax.experimental.pallas import tpu_sc as plsc

N = 10000
E = 320000
D = 128
F = 64
G = 25
GP = 32
NI = 3
NG = 64
MAXZ = 100
RC = 6.0
LOG2 = 0.6931471805599453

NC = 2
NS = 16
NW = NC * NS

CHUNK = 128
CPW = 80
E_PAD = NW * CPW * CHUNK
NBLK = 79
N_PAD = NBLK * 128
RPS = N_PAD // NS
ZR = RPS // 4
EB = 512
EBLK = E_PAD // EB


def _ssp(x):
    return jax.nn.softplus(x) - LOG2


def _node_init_body(z_ref, emb_ref, w_ref, x_ref, v_ref):
    z = z_ref[...][:, 0]
    oh = (z[:, None] == lax.broadcasted_iota(jnp.int32, (128, 128), 1))
    x = jnp.dot(oh.astype(jnp.float32), emb_ref[...],
                preferred_element_type=jnp.float32)
    x_ref[...] = x
    v_ref[...] = jnp.dot(x, w_ref[...], preferred_element_type=jnp.float32)


def _node_init(z2, emb_pad, w0):
    return pl.pallas_call(
        _node_init_body,
        grid=(NBLK,),
        in_specs=[
            pl.BlockSpec((128, 1), lambda n: (n, 0)),
            pl.BlockSpec((128, D), lambda n: (0, 0)),
            pl.BlockSpec((D, F), lambda n: (0, 0)),
        ],
        out_specs=[
            pl.BlockSpec((128, D), lambda n: (n, 0)),
            pl.BlockSpec((128, F), lambda n: (n, 0)),
        ],
        out_shape=[
            jax.ShapeDtypeStruct((N_PAD, D), jnp.float32),
            jax.ShapeDtypeStruct((N_PAD, F), jnp.float32),
        ],
    )(z2, emb_pad, w0)


def _filters_body(dr_ref, em_ref, w1_ref, b1_ref, w2_ref, b2_ref,
                  wf0_ref, wf1_ref, wf2_ref):
    dr = dr_ref[...]
    offs = lax.broadcasted_iota(jnp.int32, (1, GP), 1).astype(jnp.float32) \
        * (RC / (G - 1))
    coeff = -0.5 / (RC / (G - 1)) ** 2
    dexp = jnp.exp(coeff * (dr - offs) ** 2)
    cut = 0.5 * (jnp.cos(jnp.pi / RC * dr) + 1.0)
    cut = cut * (dr < RC).astype(jnp.float32) * em_ref[...]
    h = _ssp(jnp.dot(dexp, w1_ref[...], preferred_element_type=jnp.float32)
             + b1_ref[...])
    wf = (jnp.dot(h, w2_ref[...], preferred_element_type=jnp.float32)
          + b2_ref[...]) * cut
    wf0_ref[...] = wf[:, 0:F]
    wf1_ref[...] = wf[:, F:2 * F]
    wf2_ref[...] = wf[:, 2 * F:3 * F]


def _filters(dr2, em2, w1c, b1c, w2bd, b2c):
    return pl.pallas_call(
        _filters_body,
        grid=(EBLK,),
        in_specs=[
            pl.BlockSpec((EB, 1), lambda e: (e, 0)),
            pl.BlockSpec((EB, 1), lambda e: (e, 0)),
            pl.BlockSpec((GP, NI * F), lambda e: (0, 0)),
            pl.BlockSpec((1, NI * F), lambda e: (0, 0)),
            pl.BlockSpec((NI * F, NI * F), lambda e: (0, 0)),
            pl.BlockSpec((1, NI * F), lambda e: (0, 0)),
        ],
        out_specs=[pl.BlockSpec((EB, F), lambda e: (e, 0))] * NI,
        out_shape=[jax.ShapeDtypeStruct((E_PAD, F), jnp.float32)] * NI,
    )(dr2, em2, w1c, b1c, w2bd, b2c)


def _sc_conv_body(v_hbm, wf_hbm, s_hbm, r_hbm, out_hbm,
                  sidx, ridx, rows2, wfv2, mbuf2, zbuf, agg_sh,
                  sem, sg0, sg1, sw0, sw1, ss0, ss1):
    c = lax.axis_index("c")
    s = lax.axis_index("s")
    w = c * NS + s
    sgs, sws, sss = (sg0, sg1), (sw0, sw1), (ss0, ss1)

    cp_s = pltpu.async_copy(s_hbm.at[pl.ds(w * CPW, CPW)], sidx, sem)
    cp_r = pltpu.async_copy(r_hbm.at[pl.ds(w * CPW, CPW)], ridx, sem)

    zero16 = jnp.zeros((16,), jnp.float32)

    @pl.loop(0, ZR)
    def _(r):
        for cc in range(F // 16):
            zbuf.at[r, pl.ds(cc * 16, 16)][...] = zero16

    cp_s.wait()
    cp_r.wait()
    for t in range(RPS // ZR):
        pltpu.async_copy(zbuf, agg_sh.at[pl.ds(s * RPS + t * ZR, ZR)], sem)
    pltpu.make_async_copy(zbuf, agg_sh.at[pl.ds(0, ZR)], sem).wait()
    pltpu.make_async_copy(zbuf, agg_sh.at[pl.ds(0, ZR)], sem).wait()
    pltpu.make_async_copy(zbuf, agg_sh.at[pl.ds(0, ZR)], sem).wait()
    pltpu.make_async_copy(zbuf, agg_sh.at[pl.ds(0, ZR)], sem).wait()
    plsc.subcore_barrier()

    def start(j, b):
        base = (w * CPW + j) * CHUNK
        pltpu.async_copy(wf_hbm.at[pl.ds(base, CHUNK)], wfv2.at[b], sws[b])
        pltpu.async_copy(v_hbm.at[sidx.at[j]], rows2.at[b], sgs[b])

    def wait_in(j, b):
        pltpu.make_async_copy(wf_hbm.at[pl.ds(0, CHUNK)], wfv2.at[b],
                              sws[b]).wait()
        pltpu.make_async_copy(v_hbm.at[sidx.at[j]], rows2.at[b],
                              sgs[b]).wait()

    def mul(b):
        @pl.loop(0, CHUNK)
        def _(r):
            for cc in range(F // 16):
                sl = pl.ds(cc * 16, 16)
                mbuf2.at[b, r, sl][...] = (rows2.at[b, r, sl][...]
                                           * wfv2.at[b, r, sl][...])

    def scat(j, b):
        pltpu.async_copy(mbuf2.at[b], agg_sh.at[ridx.at[j]], sss[b], add=True)

    def wait_scat(b):
        pltpu.make_async_copy(mbuf2.at[b], agg_sh.at[ridx.at[0]],
                              sss[b]).wait()

    for b in range(2):
        start(b, b)
    for b in range(2):
        wait_in(b, b)
        mul(b)
        scat(b, b)
        start(b + 2, b)

    @pl.loop(2, CPW, step=2)
    def _(j):
        for b in range(2):
            jj = j + b
            wait_scat(b)
            wait_in(jj, b)
            mul(b)
            scat(jj, b)

            @pl.when(jj + 2 < CPW)
            def _():
                start(jj + 2, b)

    for b in range(2):
        wait_scat(b)

    plsc.subcore_barrier()
    pltpu.async_copy(agg_sh.at[pl.ds(s * RPS, RPS)],
                     out_hbm.at[c, pl.ds(s * RPS, RPS)], sem).wait()


@functools.cache
def _sc_conv_kernel():
    mesh = plsc.VectorSubcoreMesh(core_axis_name="c", subcore_axis_name="s",
                                  num_cores=NC, num_subcores=NS)
    return pl.kernel(
        _sc_conv_body,
        out_type=jax.ShapeDtypeStruct((NC, N_PAD, F), jnp.float32),
        mesh=mesh,
        compiler_params=pltpu.CompilerParams(use_tc_tiling_on_sc=False),
        scratch_types=[
            pltpu.VMEM((CPW, CHUNK), jnp.int32),
            pltpu.VMEM((CPW, CHUNK), jnp.int32),
            pltpu.VMEM((2, CHUNK, F), jnp.float32),
            pltpu.VMEM((2, CHUNK, F), jnp.float32),
            pltpu.VMEM((2, CHUNK, F), jnp.float32),
            pltpu.VMEM((ZR, F), jnp.float32),
            pltpu.VMEM_SHARED((N_PAD, F), jnp.float32),
            pltpu.SemaphoreType.DMA,
            pltpu.SemaphoreType.DMA,
            pltpu.SemaphoreType.DMA,
            pltpu.SemaphoreType.DMA,
            pltpu.SemaphoreType.DMA,
            pltpu.SemaphoreType.DMA,
            pltpu.SemaphoreType.DMA,
        ],
    )


def _sc_conv(v, wf, s2, r2):
    return _sc_conv_kernel()(v, wf, s2, r2)


def _interact_body(agg_ref, x_ref, fw_ref, fb_ref, dw_ref, db_ref, nw_ref,
                   xo_ref, vo_ref):
    agg = agg_ref[0] + agg_ref[1]
    t = _ssp(jnp.dot(agg, fw_ref[...], preferred_element_type=jnp.float32)
             + fb_ref[...])
    t = jnp.dot(t, dw_ref[...], preferred_element_type=jnp.float32) + db_ref[...]
    xn = x_ref[...] + t
    xo_ref[...] = xn
    vo_ref[...] = jnp.dot(xn, nw_ref[...], preferred_element_type=jnp.float32)


def _interact(agg2, x, fw, fb, dw, db, nw):
    return pl.pallas_call(
        _interact_body,
        grid=(NBLK,),
        in_specs=[
            pl.BlockSpec((NC, 128, F), lambda n: (0, n, 0)),
            pl.BlockSpec((128, D), lambda n: (n, 0)),
            pl.BlockSpec((F, D), lambda n: (0, 0)),
            pl.BlockSpec((1, D), lambda n: (0, 0)),
            pl.BlockSpec((D, D), lambda n: (0, 0)),
            pl.BlockSpec((1, D), lambda n: (0, 0)),
            pl.BlockSpec((D, F), lambda n: (0, 0)),
        ],
        out_specs=[
            pl.BlockSpec((128, D), lambda n: (n, 0)),
            pl.BlockSpec((128, F), lambda n: (n, 0)),
        ],
        out_shape=[
            jax.ShapeDtypeStruct((N_PAD, D), jnp.float32),
            jax.ShapeDtypeStruct((N_PAD, F), jnp.float32),
        ],
    )(agg2, x, fw, fb, dw, db, nw)


def _pool_body(x_ref, w1_ref, b1_ref, w2_ref, b2_ref, nm_ref, gi_ref, o_ref):
    h = _ssp(jnp.dot(x_ref[...], w1_ref[...],
                     preferred_element_type=jnp.float32) + b1_ref[...])
    yi = jnp.sum(h * w2_ref[...], axis=1) + b2_ref[0, 0]
    yi = yi * nm_ref[...][:, 0]
    gi = gi_ref[...][:, 0]
    oh = (gi[:, None] == lax.broadcasted_iota(jnp.int32, (128, NG), 1))
    part = jnp.sum(yi[:, None] * oh.astype(jnp.float32), axis=0)

    @pl.when(pl.program_id(0) == 0)
    def _():
        o_ref[...] = jnp.zeros_like(o_ref)

    o_ref[...] += part[None, :]


def _pool(x, w1, b1, w2r, b2, nm2, gi2):
    return pl.pallas_call(
        _pool_body,
        grid=(NBLK,),
        in_specs=[
            pl.BlockSpec((128, D), lambda n: (n, 0)),
            pl.BlockSpec((D, 32), lambda n: (0, 0)),
            pl.BlockSpec((1, 32), lambda n: (0, 0)),
            pl.BlockSpec((1, 32), lambda n: (0, 0)),
            pl.BlockSpec((1, 1), lambda n: (0, 0)),
            pl.BlockSpec((128, 1), lambda n: (n, 0)),
            pl.BlockSpec((128, 1), lambda n: (n, 0)),
        ],
        out_specs=pl.BlockSpec((1, NG), lambda n: (0, 0)),
        out_shape=jax.ShapeDtypeStruct((1, NG), jnp.float32),
    )(x, w1, b1, w2r, b2, nm2, gi2)


def kernel(z, dR, senders, receivers, graph_idx, node_mask, edge_mask,
           embed, in2f_W, fnet_W1, fnet_b1, fnet_W2, fnet_b2,
           f2out_W, f2out_b, dense_W, dense_b,
           aw_W1, aw_b1, aw_W2, aw_b2):
    f32 = jnp.float32
    z2 = jnp.pad(z.astype(jnp.int32), (0, N_PAD - N)).reshape(N_PAD, 1)
    dr2 = jnp.pad(dR, (0, E_PAD - E), constant_values=2.0 * RC) \
             .reshape(E_PAD, 1)
    em2 = jnp.pad(edge_mask, (0, E_PAD - E)).reshape(E_PAD, 1)
    s2 = jnp.pad(senders.astype(jnp.int32), (0, E_PAD - E)) \
            .reshape(NW * CPW, CHUNK)
    r2 = jnp.pad(receivers.astype(jnp.int32), (0, E_PAD - E)) \
            .reshape(NW * CPW, CHUNK)
    nm2 = jnp.pad(node_mask, (0, N_PAD - N)).reshape(N_PAD, 1)
    gi2 = jnp.pad(graph_idx.astype(jnp.int32), (0, N_PAD - N)) \
             .reshape(N_PAD, 1)

    emb_pad = jnp.pad(embed, ((0, 128 - MAXZ), (0, 0)))
    w1c = jnp.pad(jnp.concatenate([fnet_W1[i] for i in range(NI)], axis=1),
                  ((0, GP - G), (0, 0)))
    b1c = jnp.concatenate([fnet_b1[i] for i in range(NI)]).reshape(1, NI * F)
    w2bd = jnp.zeros((NI * F, NI * F), f32)
    for i in range(NI):
        w2bd = w2bd.at[i * F:(i + 1) * F, i * F:(i + 1) * F].set(fnet_W2[i])
    b2c = jnp.concatenate([fnet_b2[i] for i in range(NI)]).reshape(1, NI * F)

    x, v = _node_init(z2, emb_pad, in2f_W[0])
    wfs = _filters(dr2, em2, w1c, b1c, w2bd, b2c)
    for i in range(NI):
        agg2 = _sc_conv(v, wfs[i], s2, r2)
        nxt = in2f_W[i + 1] if i + 1 < NI else in2f_W[0]
        x, v = _interact(agg2, x, f2out_W[i], f2out_b[i].reshape(1, D),
                         dense_W[i], dense_b[i].reshape(1, D), nxt)

    out = _pool(x, aw_W1, aw_b1.reshape(1, 32), aw_W2.reshape(1, 32),
                aw_b2.reshape(1, 1), nm2, gi2)
    return out.reshape(NG)


# speedup vs baseline: 2.4595x; 1.2443x over previous
import functools

import jax
import jax.numpy as jnp
from jax import lax
from jax.experimental import pallas as pl
from jax.experimental.pallas import tpu as pltpu
from jax.experimental.pallas import tpu_sc as plsc

N = 10000
E = 320000
D = 128
F = 64
G = 25
GP = 32
NI = 3
NG = 64
MAXZ = 100
RC = 6.0
LOG2 = 0.6931471805599453

NC = 2
NS = 16
NW = NC * NS

CHUNK = 128
CPW = 80
E_PAD = NW * CPW * CHUNK
NBLK = 79
N_PAD = NBLK * 128
RPS = N_PAD // NS
ZR = 8
EB = 512
EBLK = E_PAD // EB

_TDIMS = (((0,), (0,)), ((), ()))


def _ssp(x):
    return jax.nn.softplus(x) - LOG2


def _col(row):
    one = jnp.ones((1, 1), jnp.float32)
    return lax.dot_general(row, one, _TDIMS,
                           preferred_element_type=jnp.float32)


def _evod(n):
    i2 = 2 * lax.broadcasted_iota(jnp.int32, (n, 2 * n), 0)
    j = lax.broadcasted_iota(jnp.int32, (n, 2 * n), 1)
    return (j == i2).astype(jnp.float32), (j == i2 + 1).astype(jnp.float32)


def _fold(v):
    n = v.shape[0] // 2
    ev, od = _evod(n)
    return jnp.concatenate(
        [jnp.dot(ev, v, preferred_element_type=jnp.float32),
         jnp.dot(od, v, preferred_element_type=jnp.float32)], axis=1)


def _unfold(vf):
    n = vf.shape[0]
    f = vf.shape[1] // 2
    ev, od = _evod(n)
    a = lax.dot_general(ev, vf[:, :f], _TDIMS,
                        preferred_element_type=jnp.float32)
    b = lax.dot_general(od, vf[:, f:], _TDIMS,
                        preferred_element_type=jnp.float32)
    return a + b


def _node_init_body(z_ref, emb_ref, w_ref, x_ref, v_ref):
    zr = z_ref[...].reshape(1, 128)
    k = lax.broadcasted_iota(jnp.int32, (128, 128), 0)
    oht = (k == zr).astype(jnp.float32)
    x = lax.dot_general(oht, emb_ref[...], _TDIMS,
                        preferred_element_type=jnp.float32)
    x_ref[...] = x
    v = jnp.dot(x, w_ref[...], preferred_element_type=jnp.float32)
    v_ref[...] = _fold(v)


def _node_init(z2, emb_pad, w0):
    return pl.pallas_call(
        _node_init_body,
        grid=(NBLK,),
        in_specs=[
            pl.BlockSpec((1, 1, 128), lambda n: (n, 0, 0)),
            pl.BlockSpec((128, D), lambda n: (0, 0)),
            pl.BlockSpec((D, F), lambda n: (0, 0)),
        ],
        out_specs=[
            pl.BlockSpec((128, D), lambda n: (n, 0)),
            pl.BlockSpec((64, 128), lambda n: (n, 0)),
        ],
        out_shape=[
            jax.ShapeDtypeStruct((N_PAD, D), jnp.float32),
            jax.ShapeDtypeStruct((N_PAD // 2, 128), jnp.float32),
        ],
    )(z2, emb_pad, w0)


def _filter_body(dr_ref, em_ref, w1_ref, b1_ref, w2_ref, b2_ref, wf_ref):
    drr = dr_ref[...].reshape(1, EB)
    cutr = 0.5 * (jnp.cos(jnp.pi / RC * drr) + 1.0)
    cutr = cutr * (drr < RC).astype(jnp.float32) \
        * em_ref[...].reshape(1, EB)
    dc = _col(drr)
    offs = lax.broadcasted_iota(jnp.int32, (1, GP), 1).astype(jnp.float32) \
        * (RC / (G - 1))
    coeff = -0.5 / (RC / (G - 1)) ** 2
    dexp = jnp.exp(coeff * (dc - offs) ** 2)
    h = _ssp(jnp.dot(dexp, w1_ref[...], preferred_element_type=jnp.float32)
             + b1_ref[...])
    wf = jnp.dot(h, w2_ref[...], preferred_element_type=jnp.float32) \
        + b2_ref[...]
    wf_ref[...] = wf * _col(cutr)


def _filter2(dr2, em2, w1c, b1c, w2bd, b2c):
    return pl.pallas_call(
        _filter_body,
        grid=(EBLK,),
        in_specs=[
            pl.BlockSpec((1, 1, EB), lambda e: (e, 0, 0)),
            pl.BlockSpec((1, 1, EB), lambda e: (e, 0, 0)),
            pl.BlockSpec((GP, 128), lambda e: (0, 0)),
            pl.BlockSpec((1, 128), lambda e: (0, 0)),
            pl.BlockSpec((128, 128), lambda e: (0, 0)),
            pl.BlockSpec((1, 128), lambda e: (0, 0)),
        ],
        out_specs=pl.BlockSpec((EB, 128), lambda e: (e, 0)),
        out_shape=jax.ShapeDtypeStruct((E_PAD, 128), jnp.float32),
    )(dr2, em2, w1c, b1c, w2bd, b2c)


def _sc_conv_body(off, v_hbm, wf_hbm, s_hbm, r_hbm, out_hbm,
                  sidx, ridx, rows2, wfv2, mbuf2, zbuf, agg_sh,
                  sem, sg0, sg1, sw0, sw1, ss0, ss1):
    c = lax.axis_index("c")
    s = lax.axis_index("s")
    w = c * NS + s
    sgs, sws, sss = (sg0, sg1), (sw0, sw1), (ss0, ss1)

    cp_s = pltpu.async_copy(s_hbm.at[pl.ds(w * CPW, CPW)], sidx, sem)
    cp_r = pltpu.async_copy(r_hbm.at[pl.ds(w * CPW, CPW)], ridx, sem)

    zero16 = jnp.zeros((16,), jnp.float32)

    @pl.loop(0, ZR)
    def _(r):
        for cc in range(F // 16):
            zbuf.at[r, pl.ds(cc * 16, 16)][...] = zero16

    cp_s.wait()
    cp_r.wait()

    @pl.loop(0, RPS // ZR)
    def _(t):
        pltpu.async_copy(zbuf, agg_sh.at[pl.ds(s * RPS + t * ZR, ZR)], sem)

    @pl.loop(0, RPS // ZR)
    def _(t):
        pltpu.make_async_copy(zbuf, agg_sh.at[pl.ds(0, ZR)], sem).wait()

    plsc.subcore_barrier()

    def start(j, b):
        base = (w * CPW + j) * CHUNK
        pltpu.async_copy(wf_hbm.at[pl.ds(base, CHUNK)], wfv2.at[b], sws[b])
        pltpu.async_copy(v_hbm.at[sidx.at[j]], rows2.at[b], sgs[b])

    def wait_in(j, b):
        pltpu.make_async_copy(wf_hbm.at[pl.ds(0, CHUNK)], wfv2.at[b],
                              sws[b]).wait()
        pltpu.make_async_copy(v_hbm.at[sidx.at[j]], rows2.at[b],
                              sgs[b]).wait()

    def mul(b):
        @pl.loop(0, CHUNK)
        def _(r):
            for cc in range(F // 16):
                sl = pl.ds(cc * 16, 16)
                wsl = pl.ds(off + cc * 16, 16)
                mbuf2.at[b, r, sl][...] = (rows2.at[b, r, sl][...]
                                           * wfv2.at[b, r, wsl][...])

    def scat(j, b):
        pltpu.async_copy(mbuf2.at[b], agg_sh.at[ridx.at[j]], sss[b], add=True)

    def wait_scat(b):
        pltpu.make_async_copy(mbuf2.at[b], agg_sh.at[ridx.at[0]],
                              sss[b]).wait()

    for b in range(2):
        start(b, b)
    for b in range(2):
        wait_in(b, b)
        mul(b)
        scat(b, b)
        start(b + 2, b)

    @pl.loop(2, CPW, step=2)
    def _(j):
        for b in range(2):
            jj = j + b
            wait_scat(b)
            wait_in(jj, b)
            mul(b)
            scat(jj, b)

            @pl.when(jj + 2 < CPW)
            def _():
                start(jj + 2, b)

    for b in range(2):
        wait_scat(b)

    plsc.subcore_barrier()
    pltpu.async_copy(agg_sh.at[pl.ds(s * RPS, RPS)],
                     out_hbm.at[c, pl.ds(s * RPS, RPS)], sem).wait()


@functools.cache
def _sc_conv_kernel(off):
    mesh = plsc.VectorSubcoreMesh(core_axis_name="c", subcore_axis_name="s",
                                  num_cores=NC, num_subcores=NS)
    return pl.kernel(
        functools.partial(_sc_conv_body, off),
        out_type=jax.ShapeDtypeStruct((NC, N_PAD, F), jnp.float32),
        mesh=mesh,
        compiler_params=pltpu.CompilerParams(use_tc_tiling_on_sc=False),
        scratch_types=[
            pltpu.VMEM((CPW, CHUNK), jnp.int32),
            pltpu.VMEM((CPW, CHUNK), jnp.int32),
            pltpu.VMEM((2, CHUNK, F), jnp.float32),
            pltpu.VMEM((2, CHUNK, 128), jnp.float32),
            pltpu.VMEM((2, CHUNK, F), jnp.float32),
            pltpu.VMEM((ZR, F), jnp.float32),
            pltpu.VMEM_SHARED((N_PAD, F), jnp.float32),
            pltpu.SemaphoreType.DMA,
            pltpu.SemaphoreType.DMA,
            pltpu.SemaphoreType.DMA,
            pltpu.SemaphoreType.DMA,
            pltpu.SemaphoreType.DMA,
            pltpu.SemaphoreType.DMA,
            pltpu.SemaphoreType.DMA,
        ],
    )


def _sc_conv(off, v, wf, s2, r2):
    return _sc_conv_kernel(off)(v, wf, s2, r2)


def _interact_body(agg_ref, x_ref, fw_ref, fb_ref, dw_ref, db_ref, nw_ref,
                   xo_ref, vo_ref):
    agg = _unfold(agg_ref[0] + agg_ref[1])
    t = _ssp(jnp.dot(agg, fw_ref[...], preferred_element_type=jnp.float32)
             + fb_ref[...])
    t = jnp.dot(t, dw_ref[...], preferred_element_type=jnp.float32) + db_ref[...]
    xn = x_ref[...] + t
    xo_ref[...] = xn
    v = jnp.dot(xn, nw_ref[...], preferred_element_type=jnp.float32)
    vo_ref[...] = _fold(v)


def _interact(agg2, x, fw, fb, dw, db, nw):
    return pl.pallas_call(
        _interact_body,
        grid=(NBLK,),
        in_specs=[
            pl.BlockSpec((NC, 64, 128), lambda n: (0, n, 0)),
            pl.BlockSpec((128, D), lambda n: (n, 0)),
            pl.BlockSpec((F, D), lambda n: (0, 0)),
            pl.BlockSpec((1, D), lambda n: (0, 0)),
            pl.BlockSpec((D, D), lambda n: (0, 0)),
            pl.BlockSpec((1, D), lambda n: (0, 0)),
            pl.BlockSpec((D, F), lambda n: (0, 0)),
        ],
        out_specs=[
            pl.BlockSpec((128, D), lambda n: (n, 0)),
            pl.BlockSpec((64, 128), lambda n: (n, 0)),
        ],
        out_shape=[
            jax.ShapeDtypeStruct((N_PAD, D), jnp.float32),
            jax.ShapeDtypeStruct((N_PAD // 2, 128), jnp.float32),
        ],
    )(agg2, x, fw, fb, dw, db, nw)


def _pool_body(x_ref, w1_ref, b1_ref, w2_ref, b2_ref, nm_ref, gi_ref, o_ref):
    h = _ssp(jnp.dot(x_ref[...], w1_ref[...],
                     preferred_element_type=jnp.float32) + b1_ref[...])
    yi = jnp.sum(h * w2_ref[...], axis=1) + b2_ref[0, 0]
    yi = yi * _col(nm_ref[...].reshape(1, 128))[:, 0]
    gir = gi_ref[...].reshape(1, 128)
    g = lax.broadcasted_iota(jnp.int32, (NG, 128), 0)
    oht = (g == gir).astype(jnp.float32)
    part = jnp.dot(oht, yi[:, None],
                   preferred_element_type=jnp.float32)
    eye = (lax.broadcasted_iota(jnp.int32, (NG, NG), 0)
           == lax.broadcasted_iota(jnp.int32, (NG, NG), 1)).astype(jnp.float32)
    prow = lax.dot_general(part, eye, _TDIMS,
                           preferred_element_type=jnp.float32)

    @pl.when(pl.program_id(0) == 0)
    def _():
        o_ref[...] = jnp.zeros_like(o_ref)

    o_ref[...] += prow


def _pool(x, w1, b1, w2r, b2, nm2, gi2):
    return pl.pallas_call(
        _pool_body,
        grid=(NBLK,),
        in_specs=[
            pl.BlockSpec((128, D), lambda n: (n, 0)),
            pl.BlockSpec((D, 32), lambda n: (0, 0)),
            pl.BlockSpec((1, 32), lambda n: (0, 0)),
            pl.BlockSpec((1, 32), lambda n: (0, 0)),
            pl.BlockSpec((1, 1), lambda n: (0, 0)),
            pl.BlockSpec((1, 1, 128), lambda n: (n, 0, 0)),
            pl.BlockSpec((1, 1, 128), lambda n: (n, 0, 0)),
        ],
        out_specs=pl.BlockSpec((1, NG), lambda n: (0, 0)),
        out_shape=jax.ShapeDtypeStruct((1, NG), jnp.float32),
    )(x, w1, b1, w2r, b2, nm2, gi2)


def kernel(z, dR, senders, receivers, graph_idx, node_mask, edge_mask,
           embed, in2f_W, fnet_W1, fnet_b1, fnet_W2, fnet_b2,
           f2out_W, f2out_b, dense_W, dense_b,
           aw_W1, aw_b1, aw_W2, aw_b2):
    f32 = jnp.float32
    z2 = jnp.pad(z.astype(jnp.int32), (0, N_PAD - N)).reshape(NBLK, 1, 128)
    dr2 = jnp.pad(dR, (0, E_PAD - E), constant_values=2.0 * RC) \
             .reshape(EBLK, 1, EB)
    em2 = jnp.pad(edge_mask, (0, E_PAD - E)).reshape(EBLK, 1, EB)
    s2 = jnp.pad(senders.astype(jnp.int32), (0, E_PAD - E)) \
            .reshape(NW * CPW, CHUNK)
    r2 = jnp.pad(receivers.astype(jnp.int32), (0, E_PAD - E)) \
            .reshape(NW * CPW, CHUNK)
    nm2 = jnp.pad(node_mask, (0, N_PAD - N)).reshape(NBLK, 1, 128)
    gi2 = jnp.pad(graph_idx.astype(jnp.int32), (0, N_PAD - N)) \
             .reshape(NBLK, 1, 128)

    emb_pad = jnp.pad(embed, ((0, 128 - MAXZ), (0, 0)))

    def packed_weights(ia, ib):
        w1c = jnp.pad(jnp.concatenate([fnet_W1[ia], fnet_W1[ib]], axis=1),
                      ((0, GP - G), (0, 0)))
        b1c = jnp.concatenate([fnet_b1[ia], fnet_b1[ib]]).reshape(1, 128)
        w2bd = jnp.zeros((128, 128), f32)
        w2bd = w2bd.at[:F, :F].set(fnet_W2[ia])
        w2bd = w2bd.at[F:, F:].set(fnet_W2[ib])
        b2c = jnp.concatenate([fnet_b2[ia], fnet_b2[ib]]).reshape(1, 128)
        return w1c, b1c, w2bd, b2c

    x, v128 = _node_init(z2, emb_pad, in2f_W[0])
    wf01 = _filter2(dr2, em2, *packed_weights(0, 1))
    wf22 = _filter2(dr2, em2, *packed_weights(2, 2))
    wf_off = [(wf01, 0), (wf01, F), (wf22, 0)]
    for i in range(NI):
        wf, off = wf_off[i]
        agg = _sc_conv(off, v128.reshape(N_PAD, F), wf, s2, r2)
        nxt = in2f_W[i + 1] if i + 1 < NI else in2f_W[0]
        x, v128 = _interact(agg.reshape(NC, N_PAD // 2, 128), x,
                            f2out_W[i], f2out_b[i].reshape(1, D),
                            dense_W[i], dense_b[i].reshape(1, D), nxt)

    out = _pool(x, aw_W1, aw_b1.reshape(1, 32), aw_W2.reshape(1, 32),
                aw_b2.reshape(1, 1), nm2, gi2)
    return out.reshape(NG)


# speedup vs baseline: 2.8573x; 1.1617x over previous
import functools

import jax
import jax.numpy as jnp
from jax import lax
from jax.experimental import pallas as pl
from jax.experimental.pallas import tpu as pltpu
from jax.experimental.pallas import tpu_sc as plsc

N = 10000
E = 320000
D = 128
F = 64
G = 25
GP = 32
NI = 3
NG = 64
MAXZ = 100
RC = 6.0
LOG2 = 0.6931471805599453

NC = 2
NS = 16
NW = NC * NS

CHUNK = 128
CPW = 80
E_PAD = NW * CPW * CHUNK
NBLK = 79
N_PAD = NBLK * 128
RPS = N_PAD // NS
ZR = 8
EB = 1024
EBLK = E_PAD // EB

_TDIMS = (((0,), (0,)), ((), ()))


def _ssp(x):
    return jax.nn.softplus(x) - LOG2


def _col(row):
    one = jnp.ones((1, 1), jnp.float32)
    return lax.dot_general(row, one, _TDIMS,
                           preferred_element_type=jnp.float32)


def _evod(n):
    i2 = 2 * lax.broadcasted_iota(jnp.int32, (n, 2 * n), 0)
    j = lax.broadcasted_iota(jnp.int32, (n, 2 * n), 1)
    return (j == i2).astype(jnp.float32), (j == i2 + 1).astype(jnp.float32)


def _fold(v):
    n = v.shape[0] // 2
    ev, od = _evod(n)
    return jnp.concatenate(
        [jnp.dot(ev, v, preferred_element_type=jnp.float32),
         jnp.dot(od, v, preferred_element_type=jnp.float32)], axis=1)


def _unfold(vf):
    n = vf.shape[0]
    f = vf.shape[1] // 2
    ev, od = _evod(n)
    a = lax.dot_general(ev, vf[:, :f], _TDIMS,
                        preferred_element_type=jnp.float32)
    b = lax.dot_general(od, vf[:, f:], _TDIMS,
                        preferred_element_type=jnp.float32)
    return a + b


def _node_init_body(z_ref, emb_ref, w_ref, x_ref, v_ref):
    zr = z_ref[...].reshape(1, 128)
    k = lax.broadcasted_iota(jnp.int32, (128, 128), 0)
    oht = (k == zr).astype(jnp.float32)
    x = lax.dot_general(oht, emb_ref[...], _TDIMS,
                        preferred_element_type=jnp.float32)
    x_ref[...] = x
    v = jnp.dot(x, w_ref[...], preferred_element_type=jnp.float32)
    v_ref[...] = _fold(v)


def _node_init(z2, emb_pad, w0):
    return pl.pallas_call(
        _node_init_body,
        grid=(NBLK,),
        in_specs=[
            pl.BlockSpec((1, 1, 128), lambda n: (n, 0, 0)),
            pl.BlockSpec((128, D), lambda n: (0, 0)),
            pl.BlockSpec((D, F), lambda n: (0, 0)),
        ],
        out_specs=[
            pl.BlockSpec((128, D), lambda n: (n, 0)),
            pl.BlockSpec((64, 128), lambda n: (n, 0)),
        ],
        out_shape=[
            jax.ShapeDtypeStruct((N_PAD, D), jnp.float32),
            jax.ShapeDtypeStruct((N_PAD // 2, 128), jnp.float32),
        ],
    )(z2, emb_pad, w0)


def _filter_body(dr_ref, em_ref, w1_ref, b1_ref, w2_ref, b2_ref, wf_ref):
    drr = dr_ref[...].reshape(1, EB)
    cutr = 0.5 * (jnp.cos(jnp.pi / RC * drr) + 1.0)
    cutr = cutr * (drr < RC).astype(jnp.float32) \
        * em_ref[...].reshape(1, EB)
    dc = _col(drr)
    offs = lax.broadcasted_iota(jnp.int32, (1, GP), 1).astype(jnp.float32) \
        * (RC / (G - 1))
    coeff = -0.5 / (RC / (G - 1)) ** 2
    dexp = jnp.exp(coeff * (dc - offs) ** 2)
    h = _ssp(jnp.dot(dexp, w1_ref[...], preferred_element_type=jnp.float32)
             + b1_ref[...])
    wf = jnp.dot(h, w2_ref[...], preferred_element_type=jnp.float32) \
        + b2_ref[...]
    wf_ref[...] = wf * _col(cutr)


def _filter2(dr2, em2, w1c, b1c, w2bd, b2c):
    return pl.pallas_call(
        _filter_body,
        grid=(EBLK,),
        in_specs=[
            pl.BlockSpec((1, 1, EB), lambda e: (e, 0, 0)),
            pl.BlockSpec((1, 1, EB), lambda e: (e, 0, 0)),
            pl.BlockSpec((GP, 128), lambda e: (0, 0)),
            pl.BlockSpec((1, 128), lambda e: (0, 0)),
            pl.BlockSpec((128, 128), lambda e: (0, 0)),
            pl.BlockSpec((1, 128), lambda e: (0, 0)),
        ],
        out_specs=pl.BlockSpec((EB, 128), lambda e: (e, 0)),
        out_shape=jax.ShapeDtypeStruct((E_PAD, 128), jnp.float32),
    )(dr2, em2, w1c, b1c, w2bd, b2c)


def _sc_conv_body(off, v_hbm, wf_hbm, s_hbm, r_hbm, out_hbm,
                  sidx, ridx, rows2, wfv2, mbuf2, zbuf, agg_sh,
                  sem, sg0, sg1, sw0, sw1, ss0, ss1):
    c = lax.axis_index("c")
    s = lax.axis_index("s")
    w = c * NS + s
    sgs, sws, sss = (sg0, sg1), (sw0, sw1), (ss0, ss1)

    cp_s = pltpu.async_copy(s_hbm.at[pl.ds(w * CPW, CPW)], sidx, sem)
    cp_r = pltpu.async_copy(r_hbm.at[pl.ds(w * CPW, CPW)], ridx, sem)

    zero16 = jnp.zeros((16,), jnp.float32)

    @pl.loop(0, ZR)
    def _(r):
        for cc in range(F // 16):
            zbuf.at[r, pl.ds(cc * 16, 16)][...] = zero16

    cp_s.wait()
    cp_r.wait()

    @pl.loop(0, RPS // ZR)
    def _(t):
        pltpu.async_copy(zbuf, agg_sh.at[pl.ds(s * RPS + t * ZR, ZR)], sem)

    @pl.loop(0, RPS // ZR)
    def _(t):
        pltpu.make_async_copy(zbuf, agg_sh.at[pl.ds(0, ZR)], sem).wait()

    plsc.subcore_barrier()

    def start(j, b):
        base = (w * CPW + j) * CHUNK
        pltpu.async_copy(wf_hbm.at[pl.ds(base, CHUNK)], wfv2.at[b], sws[b])
        pltpu.async_copy(v_hbm.at[sidx.at[j]], rows2.at[b], sgs[b])

    def wait_in(j, b):
        pltpu.make_async_copy(wf_hbm.at[pl.ds(0, CHUNK)], wfv2.at[b],
                              sws[b]).wait()
        pltpu.make_async_copy(v_hbm.at[sidx.at[j]], rows2.at[b],
                              sgs[b]).wait()

    def mul(b):
        @pl.loop(0, CHUNK, step=8)
        def _(r):
            for rr in range(8):
                for cc in range(F // 16):
                    sl = pl.ds(cc * 16, 16)
                    wsl = pl.ds(off + cc * 16, 16)
                    mbuf2.at[b, r + rr, sl][...] = \
                        (rows2.at[b, r + rr, sl][...]
                         * wfv2.at[b, r + rr, wsl][...])

    def scat(j, b):
        pltpu.async_copy(mbuf2.at[b], agg_sh.at[ridx.at[j]], sss[b], add=True)

    def wait_scat(b):
        pltpu.make_async_copy(mbuf2.at[b], agg_sh.at[ridx.at[0]],
                              sss[b]).wait()

    for b in range(2):
        start(b, b)
    for b in range(2):
        wait_in(b, b)
        mul(b)
        scat(b, b)
        start(b + 2, b)

    @pl.loop(2, CPW, step=2)
    def _(j):
        for b in range(2):
            jj = j + b
            wait_scat(b)
            wait_in(jj, b)
            mul(b)
            scat(jj, b)

            @pl.when(jj + 2 < CPW)
            def _():
                start(jj + 2, b)

    for b in range(2):
        wait_scat(b)

    plsc.subcore_barrier()
    pltpu.async_copy(agg_sh.at[pl.ds(s * RPS, RPS)],
                     out_hbm.at[c, pl.ds(s * RPS, RPS)], sem).wait()


@functools.cache
def _sc_conv_kernel(off):
    mesh = plsc.VectorSubcoreMesh(core_axis_name="c", subcore_axis_name="s",
                                  num_cores=NC, num_subcores=NS)
    return pl.kernel(
        functools.partial(_sc_conv_body, off),
        out_type=jax.ShapeDtypeStruct((NC, N_PAD, F), jnp.float32),
        mesh=mesh,
        compiler_params=pltpu.CompilerParams(use_tc_tiling_on_sc=False),
        scratch_types=[
            pltpu.VMEM((CPW, CHUNK), jnp.int32),
            pltpu.VMEM((CPW, CHUNK), jnp.int32),
            pltpu.VMEM((2, CHUNK, F), jnp.float32),
            pltpu.VMEM((2, CHUNK, 128), jnp.float32),
            pltpu.VMEM((2, CHUNK, F), jnp.float32),
            pltpu.VMEM((ZR, F), jnp.float32),
            pltpu.VMEM_SHARED((N_PAD, F), jnp.float32),
            pltpu.SemaphoreType.DMA,
            pltpu.SemaphoreType.DMA,
            pltpu.SemaphoreType.DMA,
            pltpu.SemaphoreType.DMA,
            pltpu.SemaphoreType.DMA,
            pltpu.SemaphoreType.DMA,
            pltpu.SemaphoreType.DMA,
        ],
    )


def _sc_conv(off, v, wf, s2, r2):
    return _sc_conv_kernel(off)(v, wf, s2, r2)


def _interact_body(agg_ref, x_ref, fw_ref, fb_ref, dw_ref, db_ref, nw_ref,
                   xo_ref, vo_ref):
    agg = _unfold(agg_ref[0] + agg_ref[1])
    t = _ssp(jnp.dot(agg, fw_ref[...], preferred_element_type=jnp.float32)
             + fb_ref[...])
    t = jnp.dot(t, dw_ref[...], preferred_element_type=jnp.float32) + db_ref[...]
    xn = x_ref[...] + t
    xo_ref[...] = xn
    v = jnp.dot(xn, nw_ref[...], preferred_element_type=jnp.float32)
    vo_ref[...] = _fold(v)


def _interact(agg2, x, fw, fb, dw, db, nw):
    return pl.pallas_call(
        _interact_body,
        grid=(NBLK,),
        in_specs=[
            pl.BlockSpec((NC, 64, 128), lambda n: (0, n, 0)),
            pl.BlockSpec((128, D), lambda n: (n, 0)),
            pl.BlockSpec((F, D), lambda n: (0, 0)),
            pl.BlockSpec((1, D), lambda n: (0, 0)),
            pl.BlockSpec((D, D), lambda n: (0, 0)),
            pl.BlockSpec((1, D), lambda n: (0, 0)),
            pl.BlockSpec((D, F), lambda n: (0, 0)),
        ],
        out_specs=[
            pl.BlockSpec((128, D), lambda n: (n, 0)),
            pl.BlockSpec((64, 128), lambda n: (n, 0)),
        ],
        out_shape=[
            jax.ShapeDtypeStruct((N_PAD, D), jnp.float32),
            jax.ShapeDtypeStruct((N_PAD // 2, 128), jnp.float32),
        ],
    )(agg2, x, fw, fb, dw, db, nw)


def _pool_body(x_ref, w1_ref, b1_ref, w2_ref, b2_ref, nm_ref, gi_ref, o_ref):
    h = _ssp(jnp.dot(x_ref[...], w1_ref[...],
                     preferred_element_type=jnp.float32) + b1_ref[...])
    yi = jnp.sum(h * w2_ref[...], axis=1) + b2_ref[0, 0]
    yi = yi * _col(nm_ref[...].reshape(1, 128))[:, 0]
    gir = gi_ref[...].reshape(1, 128)
    g = lax.broadcasted_iota(jnp.int32, (NG, 128), 0)
    oht = (g == gir).astype(jnp.float32)
    part = jnp.dot(oht, yi[:, None],
                   preferred_element_type=jnp.float32)
    eye = (lax.broadcasted_iota(jnp.int32, (NG, NG), 0)
           == lax.broadcasted_iota(jnp.int32, (NG, NG), 1)).astype(jnp.float32)
    prow = lax.dot_general(part, eye, _TDIMS,
                           preferred_element_type=jnp.float32)

    @pl.when(pl.program_id(0) == 0)
    def _():
        o_ref[...] = jnp.zeros_like(o_ref)

    o_ref[...] += prow


def _pool(x, w1, b1, w2r, b2, nm2, gi2):
    return pl.pallas_call(
        _pool_body,
        grid=(NBLK,),
        in_specs=[
            pl.BlockSpec((128, D), lambda n: (n, 0)),
            pl.BlockSpec((D, 32), lambda n: (0, 0)),
            pl.BlockSpec((1, 32), lambda n: (0, 0)),
            pl.BlockSpec((1, 32), lambda n: (0, 0)),
            pl.BlockSpec((1, 1), lambda n: (0, 0)),
            pl.BlockSpec((1, 1, 128), lambda n: (n, 0, 0)),
            pl.BlockSpec((1, 1, 128), lambda n: (n, 0, 0)),
        ],
        out_specs=pl.BlockSpec((1, NG), lambda n: (0, 0)),
        out_shape=jax.ShapeDtypeStruct((1, NG), jnp.float32),
    )(x, w1, b1, w2r, b2, nm2, gi2)


def kernel(z, dR, senders, receivers, graph_idx, node_mask, edge_mask,
           embed, in2f_W, fnet_W1, fnet_b1, fnet_W2, fnet_b2,
           f2out_W, f2out_b, dense_W, dense_b,
           aw_W1, aw_b1, aw_W2, aw_b2):
    f32 = jnp.float32
    z2 = jnp.pad(z.astype(jnp.int32), (0, N_PAD - N)).reshape(NBLK, 1, 128)
    dr2 = jnp.pad(dR, (0, E_PAD - E), constant_values=2.0 * RC) \
             .reshape(EBLK, 1, EB)
    em2 = jnp.pad(edge_mask, (0, E_PAD - E)).reshape(EBLK, 1, EB)
    s2 = jnp.pad(senders.astype(jnp.int32), (0, E_PAD - E)) \
            .reshape(NW * CPW, CHUNK)
    r2 = jnp.pad(receivers.astype(jnp.int32), (0, E_PAD - E)) \
            .reshape(NW * CPW, CHUNK)
    nm2 = jnp.pad(node_mask, (0, N_PAD - N)).reshape(NBLK, 1, 128)
    gi2 = jnp.pad(graph_idx.astype(jnp.int32), (0, N_PAD - N)) \
             .reshape(NBLK, 1, 128)

    emb_pad = jnp.pad(embed, ((0, 128 - MAXZ), (0, 0)))

    def packed_weights(ia, ib):
        w1c = jnp.pad(jnp.concatenate([fnet_W1[ia], fnet_W1[ib]], axis=1),
                      ((0, GP - G), (0, 0)))
        b1c = jnp.concatenate([fnet_b1[ia], fnet_b1[ib]]).reshape(1, 128)
        w2bd = jnp.zeros((128, 128), f32)
        w2bd = w2bd.at[:F, :F].set(fnet_W2[ia])
        w2bd = w2bd.at[F:, F:].set(fnet_W2[ib])
        b2c = jnp.concatenate([fnet_b2[ia], fnet_b2[ib]]).reshape(1, 128)
        return w1c, b1c, w2bd, b2c

    x, v128 = _node_init(z2, emb_pad, in2f_W[0])
    wf01 = _filter2(dr2, em2, *packed_weights(0, 1))
    wf22 = _filter2(dr2, em2, *packed_weights(2, 2))
    wf_off = [(wf01, 0), (wf01, F), (wf22, 0)]
    for i in range(NI):
        wf, off = wf_off[i]
        agg = _sc_conv(off, v128.reshape(N_PAD, F), wf, s2, r2)
        nxt = in2f_W[i + 1] if i + 1 < NI else in2f_W[0]
        x, v128 = _interact(agg.reshape(NC, N_PAD // 2, 128), x,
                            f2out_W[i], f2out_b[i].reshape(1, D),
                            dense_W[i], dense_b[i].reshape(1, D), nxt)

    out = _pool(x, aw_W1, aw_b1.reshape(1, 32), aw_W2.reshape(1, 32),
                aw_b2.reshape(1, 1), nm2, gi2)
    return out.reshape(NG)


# speedup vs baseline: 3.0042x; 1.0514x over previous
import functools

import jax
import jax.numpy as jnp
from jax import lax
from jax.experimental import pallas as pl
from jax.experimental.pallas import tpu as pltpu
from jax.experimental.pallas import tpu_sc as plsc

N = 10000
E = 320000
D = 128
F = 64
G = 25
GP = 32
NI = 3
NG = 64
MAXZ = 100
RC = 6.0
LOG2 = 0.6931471805599453

NC = 2
NS = 16
NW = NC * NS

CHUNK = 128
CPW = 80
E_PAD = NW * CPW * CHUNK
NBLK = 79
N_PAD = NBLK * 128
RPS = N_PAD // NS
ZR = 8
EB = 1024
EBLK = E_PAD // EB

_TDIMS = (((0,), (0,)), ((), ()))


def _ssp(x):
    return jax.nn.softplus(x) - LOG2


def _col(row):
    one = jnp.ones((1, 1), jnp.float32)
    return lax.dot_general(row, one, _TDIMS,
                           preferred_element_type=jnp.float32)


def _evod(n):
    i2 = 2 * lax.broadcasted_iota(jnp.int32, (n, 2 * n), 0)
    j = lax.broadcasted_iota(jnp.int32, (n, 2 * n), 1)
    return (j == i2).astype(jnp.float32), (j == i2 + 1).astype(jnp.float32)


def _fold(v):
    n = v.shape[0] // 2
    ev, od = _evod(n)
    return jnp.concatenate(
        [jnp.dot(ev, v, preferred_element_type=jnp.float32),
         jnp.dot(od, v, preferred_element_type=jnp.float32)], axis=1)


def _unfold(vf):
    n = vf.shape[0]
    f = vf.shape[1] // 2
    ev, od = _evod(n)
    a = lax.dot_general(ev, vf[:, :f], _TDIMS,
                        preferred_element_type=jnp.float32)
    b = lax.dot_general(od, vf[:, f:], _TDIMS,
                        preferred_element_type=jnp.float32)
    return a + b


def _node_init_body(z_ref, emb_ref, w_ref, x_ref, v_ref):
    zr = z_ref[...].reshape(1, 128)
    k = lax.broadcasted_iota(jnp.int32, (128, 128), 0)
    oht = (k == zr).astype(jnp.float32)
    x = lax.dot_general(oht, emb_ref[...], _TDIMS,
                        preferred_element_type=jnp.float32)
    x_ref[...] = x
    v = jnp.dot(x, w_ref[...], preferred_element_type=jnp.float32)
    v_ref[...] = _fold(v)


def _node_init(z2, emb_pad, w0):
    return pl.pallas_call(
        _node_init_body,
        grid=(NBLK,),
        in_specs=[
            pl.BlockSpec((1, 1, 128), lambda n: (n, 0, 0)),
            pl.BlockSpec((128, D), lambda n: (0, 0)),
            pl.BlockSpec((D, F), lambda n: (0, 0)),
        ],
        out_specs=[
            pl.BlockSpec((128, D), lambda n: (n, 0)),
            pl.BlockSpec((64, 128), lambda n: (n, 0)),
        ],
        out_shape=[
            jax.ShapeDtypeStruct((N_PAD, D), jnp.float32),
            jax.ShapeDtypeStruct((N_PAD // 2, 128), jnp.float32),
        ],
    )(z2, emb_pad, w0)


def _filter_body(dr_ref, em_ref, w1_ref, b1_ref, w2_ref, b2_ref, wf_ref):
    drr = dr_ref[...].reshape(1, EB)
    cutr = 0.5 * (jnp.cos(jnp.pi / RC * drr) + 1.0)
    cutr = cutr * (drr < RC).astype(jnp.float32) \
        * em_ref[...].reshape(1, EB)
    dc = _col(drr)
    offs = lax.broadcasted_iota(jnp.int32, (1, GP), 1).astype(jnp.float32) \
        * (RC / (G - 1))
    coeff = -0.5 / (RC / (G - 1)) ** 2
    dexp = jnp.exp(coeff * (dc - offs) ** 2)
    h = _ssp(jnp.dot(dexp, w1_ref[...], preferred_element_type=jnp.float32)
             + b1_ref[...])
    wf = jnp.dot(h, w2_ref[...], preferred_element_type=jnp.float32) \
        + b2_ref[...]
    wf = wf * _col(cutr)
    wf_ref[...] = jnp.concatenate([wf[:EB // 2], wf[EB // 2:]], axis=1)


def _filter1(dr2, em2, w1p, b1, w2, b2):
    return pl.pallas_call(
        _filter_body,
        grid=(EBLK,),
        in_specs=[
            pl.BlockSpec((1, 1, EB), lambda e: (e, 0, 0)),
            pl.BlockSpec((1, 1, EB), lambda e: (e, 0, 0)),
            pl.BlockSpec((GP, F), lambda e: (0, 0)),
            pl.BlockSpec((1, F), lambda e: (0, 0)),
            pl.BlockSpec((F, F), lambda e: (0, 0)),
            pl.BlockSpec((1, F), lambda e: (0, 0)),
        ],
        out_specs=pl.BlockSpec((EB // 2, 128), lambda e: (e, 0)),
        out_shape=jax.ShapeDtypeStruct((E_PAD // 2, 128), jnp.float32),
    )(dr2, em2, w1p, b1, w2, b2)


def _sc_conv_body(v_hbm, wf_hbm, s_hbm, r_hbm, out_hbm,
                  sidx, ridx, rows2, wfv2, mbuf2, zbuf, agg_sh,
                  sem, sg0, sg1, sw0, sw1, ss0, ss1):
    c = lax.axis_index("c")
    s = lax.axis_index("s")
    w = c * NS + s
    sgs, sws, sss = (sg0, sg1), (sw0, sw1), (ss0, ss1)

    cp_s = pltpu.async_copy(s_hbm.at[pl.ds(w * CPW, CPW)], sidx, sem)
    cp_r = pltpu.async_copy(r_hbm.at[pl.ds(w * CPW, CPW)], ridx, sem)

    zero16 = jnp.zeros((16,), jnp.float32)

    @pl.loop(0, ZR)
    def _(r):
        for cc in range(F // 16):
            zbuf.at[r, pl.ds(cc * 16, 16)][...] = zero16

    cp_s.wait()
    cp_r.wait()

    @pl.loop(0, RPS // ZR)
    def _(t):
        pltpu.async_copy(zbuf, agg_sh.at[pl.ds(s * RPS + t * ZR, ZR)], sem)

    @pl.loop(0, RPS // ZR)
    def _(t):
        pltpu.make_async_copy(zbuf, agg_sh.at[pl.ds(0, ZR)], sem).wait()

    plsc.subcore_barrier()

    def start(j, b):
        base = (w * CPW + j) * CHUNK
        pltpu.async_copy(wf_hbm.at[pl.ds(base, CHUNK)], wfv2.at[b], sws[b])
        pltpu.async_copy(v_hbm.at[sidx.at[j]], rows2.at[b], sgs[b])

    def wait_in(j, b):
        pltpu.make_async_copy(wf_hbm.at[pl.ds(0, CHUNK)], wfv2.at[b],
                              sws[b]).wait()
        pltpu.make_async_copy(v_hbm.at[sidx.at[j]], rows2.at[b],
                              sgs[b]).wait()

    def mul(b):
        @pl.loop(0, CHUNK, step=8)
        def _(r):
            for rr in range(8):
                for cc in range(F // 16):
                    sl = pl.ds(cc * 16, 16)
                    mbuf2.at[b, r + rr, sl][...] = \
                        (rows2.at[b, r + rr, sl][...]
                         * wfv2.at[b, r + rr, sl][...])

    def scat(j, b):
        pltpu.async_copy(mbuf2.at[b], agg_sh.at[ridx.at[j]], sss[b], add=True)

    def wait_scat(b):
        pltpu.make_async_copy(mbuf2.at[b], agg_sh.at[ridx.at[0]],
                              sss[b]).wait()

    for b in range(2):
        start(b, b)
    for b in range(2):
        wait_in(b, b)
        mul(b)
        scat(b, b)
        start(b + 2, b)

    @pl.loop(2, CPW, step=2)
    def _(j):
        for b in range(2):
            jj = j + b
            wait_scat(b)
            wait_in(jj, b)
            mul(b)
            scat(jj, b)

            @pl.when(jj + 2 < CPW)
            def _():
                start(jj + 2, b)

    for b in range(2):
        wait_scat(b)

    plsc.subcore_barrier()
    pltpu.async_copy(agg_sh.at[pl.ds(s * RPS, RPS)],
                     out_hbm.at[c, pl.ds(s * RPS, RPS)], sem).wait()


@functools.cache
def _sc_conv_kernel():
    mesh = plsc.VectorSubcoreMesh(core_axis_name="c", subcore_axis_name="s",
                                  num_cores=NC, num_subcores=NS)
    return pl.kernel(
        _sc_conv_body,
        out_type=jax.ShapeDtypeStruct((NC, N_PAD, F), jnp.float32),
        mesh=mesh,
        compiler_params=pltpu.CompilerParams(use_tc_tiling_on_sc=False),
        scratch_types=[
            pltpu.VMEM((CPW, CHUNK), jnp.int32),
            pltpu.VMEM((CPW, CHUNK), jnp.int32),
            pltpu.VMEM((2, CHUNK, F), jnp.float32),
            pltpu.VMEM((2, CHUNK, F), jnp.float32),
            pltpu.VMEM((2, CHUNK, F), jnp.float32),
            pltpu.VMEM((ZR, F), jnp.float32),
            pltpu.VMEM_SHARED((N_PAD, F), jnp.float32),
            pltpu.SemaphoreType.DMA,
            pltpu.SemaphoreType.DMA,
            pltpu.SemaphoreType.DMA,
            pltpu.SemaphoreType.DMA,
            pltpu.SemaphoreType.DMA,
            pltpu.SemaphoreType.DMA,
            pltpu.SemaphoreType.DMA,
        ],
    )


def _sc_conv(v, wf, s2, r2):
    return _sc_conv_kernel()(v, wf, s2, r2)


def _interact_body(agg_ref, x_ref, fw_ref, fb_ref, dw_ref, db_ref, nw_ref,
                   xo_ref, vo_ref):
    agg = _unfold(agg_ref[0] + agg_ref[1])
    t = _ssp(jnp.dot(agg, fw_ref[...], preferred_element_type=jnp.float32)
             + fb_ref[...])
    t = jnp.dot(t, dw_ref[...], preferred_element_type=jnp.float32) + db_ref[...]
    xn = x_ref[...] + t
    xo_ref[...] = xn
    v = jnp.dot(xn, nw_ref[...], preferred_element_type=jnp.float32)
    vo_ref[...] = _fold(v)


def _interact(agg2, x, fw, fb, dw, db, nw):
    return pl.pallas_call(
        _interact_body,
        grid=(NBLK,),
        in_specs=[
            pl.BlockSpec((NC, 64, 128), lambda n: (0, n, 0)),
            pl.BlockSpec((128, D), lambda n: (n, 0)),
            pl.BlockSpec((F, D), lambda n: (0, 0)),
            pl.BlockSpec((1, D), lambda n: (0, 0)),
            pl.BlockSpec((D, D), lambda n: (0, 0)),
            pl.BlockSpec((1, D), lambda n: (0, 0)),
            pl.BlockSpec((D, F), lambda n: (0, 0)),
        ],
        out_specs=[
            pl.BlockSpec((128, D), lambda n: (n, 0)),
            pl.BlockSpec((64, 128), lambda n: (n, 0)),
        ],
        out_shape=[
            jax.ShapeDtypeStruct((N_PAD, D), jnp.float32),
            jax.ShapeDtypeStruct((N_PAD // 2, 128), jnp.float32),
        ],
    )(agg2, x, fw, fb, dw, db, nw)


def _ipool_body(agg_ref, x_ref, fw_ref, fb_ref, dw_ref, db_ref,
                w1_ref, b1_ref, w2_ref, b2_ref, nm_ref, gi_ref, o_ref):
    agg = _unfold(agg_ref[0] + agg_ref[1])
    t = _ssp(jnp.dot(agg, fw_ref[...], preferred_element_type=jnp.float32)
             + fb_ref[...])
    t = jnp.dot(t, dw_ref[...], preferred_element_type=jnp.float32) + db_ref[...]
    xn = x_ref[...] + t
    h = _ssp(jnp.dot(xn, w1_ref[...],
                     preferred_element_type=jnp.float32) + b1_ref[...])
    yi = jnp.sum(h * w2_ref[...], axis=1) + b2_ref[0, 0]
    yi = yi * _col(nm_ref[...].reshape(1, 128))[:, 0]
    gir = gi_ref[...].reshape(1, 128)
    g = lax.broadcasted_iota(jnp.int32, (NG, 128), 0)
    oht = (g == gir).astype(jnp.float32)
    part = jnp.dot(oht, yi[:, None],
                   preferred_element_type=jnp.float32)
    eye = (lax.broadcasted_iota(jnp.int32, (NG, NG), 0)
           == lax.broadcasted_iota(jnp.int32, (NG, NG), 1)).astype(jnp.float32)
    prow = lax.dot_general(part, eye, _TDIMS,
                           preferred_element_type=jnp.float32)

    @pl.when(pl.program_id(0) == 0)
    def _():
        o_ref[...] = jnp.zeros_like(o_ref)

    o_ref[...] += prow


def _ipool(agg2, x, fw, fb, dw, db, w1, b1, w2r, b2, nm2, gi2):
    return pl.pallas_call(
        _ipool_body,
        grid=(NBLK,),
        in_specs=[
            pl.BlockSpec((NC, 64, 128), lambda n: (0, n, 0)),
            pl.BlockSpec((128, D), lambda n: (n, 0)),
            pl.BlockSpec((F, D), lambda n: (0, 0)),
            pl.BlockSpec((1, D), lambda n: (0, 0)),
            pl.BlockSpec((D, D), lambda n: (0, 0)),
            pl.BlockSpec((1, D), lambda n: (0, 0)),
            pl.BlockSpec((D, 32), lambda n: (0, 0)),
            pl.BlockSpec((1, 32), lambda n: (0, 0)),
            pl.BlockSpec((1, 32), lambda n: (0, 0)),
            pl.BlockSpec((1, 1), lambda n: (0, 0)),
            pl.BlockSpec((1, 1, 128), lambda n: (n, 0, 0)),
            pl.BlockSpec((1, 1, 128), lambda n: (n, 0, 0)),
        ],
        out_specs=pl.BlockSpec((1, NG), lambda n: (0, 0)),
        out_shape=jax.ShapeDtypeStruct((1, NG), jnp.float32),
    )(agg2, x, fw, fb, dw, db, w1, b1, w2r, b2, nm2, gi2)


def _pool_body(x_ref, w1_ref, b1_ref, w2_ref, b2_ref, nm_ref, gi_ref, o_ref):
    h = _ssp(jnp.dot(x_ref[...], w1_ref[...],
                     preferred_element_type=jnp.float32) + b1_ref[...])
    yi = jnp.sum(h * w2_ref[...], axis=1) + b2_ref[0, 0]
    yi = yi * _col(nm_ref[...].reshape(1, 128))[:, 0]
    gir = gi_ref[...].reshape(1, 128)
    g = lax.broadcasted_iota(jnp.int32, (NG, 128), 0)
    oht = (g == gir).astype(jnp.float32)
    part = jnp.dot(oht, yi[:, None],
                   preferred_element_type=jnp.float32)
    eye = (lax.broadcasted_iota(jnp.int32, (NG, NG), 0)
           == lax.broadcasted_iota(jnp.int32, (NG, NG), 1)).astype(jnp.float32)
    prow = lax.dot_general(part, eye, _TDIMS,
                           preferred_element_type=jnp.float32)

    @pl.when(pl.program_id(0) == 0)
    def _():
        o_ref[...] = jnp.zeros_like(o_ref)

    o_ref[...] += prow


def _pool(x, w1, b1, w2r, b2, nm2, gi2):
    return pl.pallas_call(
        _pool_body,
        grid=(NBLK,),
        in_specs=[
            pl.BlockSpec((128, D), lambda n: (n, 0)),
            pl.BlockSpec((D, 32), lambda n: (0, 0)),
            pl.BlockSpec((1, 32), lambda n: (0, 0)),
            pl.BlockSpec((1, 32), lambda n: (0, 0)),
            pl.BlockSpec((1, 1), lambda n: (0, 0)),
            pl.BlockSpec((1, 1, 128), lambda n: (n, 0, 0)),
            pl.BlockSpec((1, 1, 128), lambda n: (n, 0, 0)),
        ],
        out_specs=pl.BlockSpec((1, NG), lambda n: (0, 0)),
        out_shape=jax.ShapeDtypeStruct((1, NG), jnp.float32),
    )(x, w1, b1, w2r, b2, nm2, gi2)


def kernel(z, dR, senders, receivers, graph_idx, node_mask, edge_mask,
           embed, in2f_W, fnet_W1, fnet_b1, fnet_W2, fnet_b2,
           f2out_W, f2out_b, dense_W, dense_b,
           aw_W1, aw_b1, aw_W2, aw_b2):
    f32 = jnp.float32
    z2 = jnp.pad(z.astype(jnp.int32), (0, N_PAD - N)).reshape(NBLK, 1, 128)
    dr2 = jnp.pad(dR, (0, E_PAD - E), constant_values=2.0 * RC) \
             .reshape(EBLK, 1, EB)
    em2 = jnp.pad(edge_mask, (0, E_PAD - E)).reshape(EBLK, 1, EB)
    nm2 = jnp.pad(node_mask, (0, N_PAD - N)).reshape(NBLK, 1, 128)
    gi2 = jnp.pad(graph_idx.astype(jnp.int32), (0, N_PAD - N)) \
             .reshape(NBLK, 1, 128)

    emb_pad = jnp.pad(embed, ((0, 128 - MAXZ), (0, 0)))
    w1p = [jnp.pad(fnet_W1[i], ((0, GP - G), (0, 0))) for i in range(NI)]

    ar = jnp.arange(E_PAD, dtype=jnp.int32)
    rb = ar % EB
    perm = (ar // EB) * EB + (rb % 2) * (EB // 2) + rb // 2
    s2 = jnp.pad(senders.astype(jnp.int32), (0, E_PAD - E))[perm] \
            .reshape(NW * CPW, CHUNK)
    r2 = jnp.pad(receivers.astype(jnp.int32), (0, E_PAD - E))[perm] \
            .reshape(NW * CPW, CHUNK)

    x, v128 = _node_init(z2, emb_pad, in2f_W[0])
    wfs = [_filter1(dr2, em2, w1p[i], fnet_b1[i].reshape(1, F),
                    fnet_W2[i], fnet_b2[i].reshape(1, F)) for i in range(NI)]
    for i in range(NI - 1):
        agg = _sc_conv(v128.reshape(N_PAD, F), wfs[i].reshape(E_PAD, F),
                       s2, r2)
        x, v128 = _interact(agg.reshape(NC, N_PAD // 2, 128), x,
                            f2out_W[i], f2out_b[i].reshape(1, D),
                            dense_W[i], dense_b[i].reshape(1, D),
                            in2f_W[i + 1])
    agg = _sc_conv(v128.reshape(N_PAD, F), wfs[NI - 1].reshape(E_PAD, F),
                   s2, r2)
    out = _ipool(agg.reshape(NC, N_PAD // 2, 128), x,
                 f2out_W[NI - 1], f2out_b[NI - 1].reshape(1, D),
                 dense_W[NI - 1], dense_b[NI - 1].reshape(1, D),
                 aw_W1, aw_b1.reshape(1, 32), aw_W2.reshape(1, 32),
                 aw_b2.reshape(1, 1), nm2, gi2)
    return out.reshape(NG)


# speedup vs baseline: 3.1491x; 1.0482x over previous
import functools

import jax
import jax.numpy as jnp
from jax import lax
from jax.experimental import pallas as pl
from jax.experimental.pallas import tpu as pltpu
from jax.experimental.pallas import tpu_sc as plsc

N = 10000
E = 320000
D = 128
F = 64
G = 25
GP = 32
NI = 3
NG = 64
MAXZ = 100
RC = 6.0
LOG2 = 0.6931471805599453

NC = 2
NS = 16
NW = NC * NS

CHUNK = 128
CPW = 80
E_PAD = NW * CPW * CHUNK
NBLK = 79
N_PAD = NBLK * 128
RPS = N_PAD // NS
ZR = 8
EB = 2048
EBLK = E_PAD // EB

_TDIMS = (((0,), (0,)), ((), ()))


def _ssp(x):
    return jax.nn.softplus(x) - LOG2


def _col(row):
    one = jnp.ones((1, 1), jnp.float32)
    return lax.dot_general(row, one, _TDIMS,
                           preferred_element_type=jnp.float32)


def _evod(n):
    i2 = 2 * lax.broadcasted_iota(jnp.int32, (n, 2 * n), 0)
    j = lax.broadcasted_iota(jnp.int32, (n, 2 * n), 1)
    return (j == i2).astype(jnp.float32), (j == i2 + 1).astype(jnp.float32)


def _fold(v):
    n = v.shape[0] // 2
    ev, od = _evod(n)
    return jnp.concatenate(
        [jnp.dot(ev, v, preferred_element_type=jnp.float32),
         jnp.dot(od, v, preferred_element_type=jnp.float32)], axis=1)


def _unfold(vf):
    n = vf.shape[0]
    f = vf.shape[1] // 2
    ev, od = _evod(n)
    a = lax.dot_general(ev, vf[:, :f], _TDIMS,
                        preferred_element_type=jnp.float32)
    b = lax.dot_general(od, vf[:, f:], _TDIMS,
                        preferred_element_type=jnp.float32)
    return a + b


def _node_init_body(z_ref, emb_ref, w_ref, x_ref, v_ref):
    zr = z_ref[...].reshape(1, 128)
    k = lax.broadcasted_iota(jnp.int32, (128, 128), 0)
    oht = (k == zr).astype(jnp.float32)
    x = lax.dot_general(oht, emb_ref[...], _TDIMS,
                        preferred_element_type=jnp.float32)
    x_ref[...] = x
    v = jnp.dot(x, w_ref[...], preferred_element_type=jnp.float32)
    v_ref[...] = _fold(v)


def _node_init(z2, emb_pad, w0):
    return pl.pallas_call(
        _node_init_body,
        grid=(NBLK,),
        in_specs=[
            pl.BlockSpec((1, 1, 128), lambda n: (n, 0, 0)),
            pl.BlockSpec((128, D), lambda n: (0, 0)),
            pl.BlockSpec((D, F), lambda n: (0, 0)),
        ],
        out_specs=[
            pl.BlockSpec((128, D), lambda n: (n, 0)),
            pl.BlockSpec((64, 128), lambda n: (n, 0)),
        ],
        out_shape=[
            jax.ShapeDtypeStruct((N_PAD, D), jnp.float32),
            jax.ShapeDtypeStruct((N_PAD // 2, 128), jnp.float32),
        ],
    )(z2, emb_pad, w0)


def _filter_body(dr_ref, em_ref, w1_ref, b1_ref, w2_ref, b2_ref, wf_ref):
    drr = dr_ref[...].reshape(1, EB)
    cutr = 0.5 * (jnp.cos(jnp.pi / RC * drr) + 1.0)
    cutr = cutr * (drr < RC).astype(jnp.float32) \
        * em_ref[...].reshape(1, EB)
    dc = _col(drr)
    offs = lax.broadcasted_iota(jnp.int32, (1, GP), 1).astype(jnp.float32) \
        * (RC / (G - 1))
    coeff = -0.5 / (RC / (G - 1)) ** 2
    dexp = jnp.exp(coeff * (dc - offs) ** 2)
    h = _ssp(jnp.dot(dexp, w1_ref[...], preferred_element_type=jnp.float32)
             + b1_ref[...])
    wf = jnp.dot(h, w2_ref[...], preferred_element_type=jnp.float32) \
        + b2_ref[...]
    wf = wf * _col(cutr)
    wf_ref[...] = jnp.concatenate([wf[:EB // 2], wf[EB // 2:]], axis=1)


def _filter1(dr2, em2, w1p, b1, w2, b2):
    return pl.pallas_call(
        _filter_body,
        grid=(EBLK,),
        in_specs=[
            pl.BlockSpec((1, 1, EB), lambda e: (e, 0, 0)),
            pl.BlockSpec((1, 1, EB), lambda e: (e, 0, 0)),
            pl.BlockSpec((GP, F), lambda e: (0, 0)),
            pl.BlockSpec((1, F), lambda e: (0, 0)),
            pl.BlockSpec((F, F), lambda e: (0, 0)),
            pl.BlockSpec((1, F), lambda e: (0, 0)),
        ],
        out_specs=pl.BlockSpec((EB // 2, 128), lambda e: (e, 0)),
        out_shape=jax.ShapeDtypeStruct((E_PAD // 2, 128), jnp.float32),
    )(dr2, em2, w1p, b1, w2, b2)


def _sc_conv_body(v_hbm, wf_hbm, s_hbm, r_hbm, out_hbm,
                  sidx, ridx, rows2, wfv2, mbuf2, zbuf, agg_sh,
                  sem, sg0, sg1, sw0, sw1, ss0, ss1):
    c = lax.axis_index("c")
    s = lax.axis_index("s")
    w = c * NS + s
    sgs, sws, sss = (sg0, sg1), (sw0, sw1), (ss0, ss1)

    cp_s = pltpu.async_copy(s_hbm.at[pl.ds(w * CPW, CPW)], sidx, sem)
    cp_r = pltpu.async_copy(r_hbm.at[pl.ds(w * CPW, CPW)], ridx, sem)

    zero16 = jnp.zeros((16,), jnp.float32)

    @pl.loop(0, ZR)
    def _(r):
        for cc in range(F // 16):
            zbuf.at[r, pl.ds(cc * 16, 16)][...] = zero16

    cp_s.wait()
    cp_r.wait()

    @pl.loop(0, RPS // ZR)
    def _(t):
        pltpu.async_copy(zbuf, agg_sh.at[pl.ds(s * RPS + t * ZR, ZR)], sem)

    @pl.loop(0, RPS // ZR)
    def _(t):
        pltpu.make_async_copy(zbuf, agg_sh.at[pl.ds(0, ZR)], sem).wait()

    plsc.subcore_barrier()

    def start(j, b):
        base = (w * CPW + j) * CHUNK
        pltpu.async_copy(wf_hbm.at[pl.ds(base, CHUNK)], wfv2.at[b], sws[b])
        pltpu.async_copy(v_hbm.at[sidx.at[j]], rows2.at[b], sgs[b])

    def wait_in(j, b):
        pltpu.make_async_copy(wf_hbm.at[pl.ds(0, CHUNK)], wfv2.at[b],
                              sws[b]).wait()
        pltpu.make_async_copy(v_hbm.at[sidx.at[j]], rows2.at[b],
                              sgs[b]).wait()

    def mul(b):
        @pl.loop(0, CHUNK, step=8)
        def _(r):
            for rr in range(8):
                for cc in range(F // 16):
                    sl = pl.ds(cc * 16, 16)
                    mbuf2.at[b, r + rr, sl][...] = \
                        (rows2.at[b, r + rr, sl][...]
                         * wfv2.at[b, r + rr, sl][...])

    def scat(j, b):
        pltpu.async_copy(mbuf2.at[b], agg_sh.at[ridx.at[j]], sss[b], add=True)

    def wait_scat(b):
        pltpu.make_async_copy(mbuf2.at[b], agg_sh.at[ridx.at[0]],
                              sss[b]).wait()

    for b in range(2):
        start(b, b)
    for b in range(2):
        wait_in(b, b)
        mul(b)
        scat(b, b)
        start(b + 2, b)

    @pl.loop(2, CPW, step=2)
    def _(j):
        for b in range(2):
            jj = j + b
            wait_scat(b)
            wait_in(jj, b)
            mul(b)
            scat(jj, b)

            @pl.when(jj + 2 < CPW)
            def _():
                start(jj + 2, b)

    for b in range(2):
        wait_scat(b)

    plsc.subcore_barrier()
    pltpu.async_copy(agg_sh.at[pl.ds(s * RPS, RPS)],
                     out_hbm.at[c, pl.ds(s * RPS, RPS)], sem).wait()


@functools.cache
def _sc_conv_kernel():
    mesh = plsc.VectorSubcoreMesh(core_axis_name="c", subcore_axis_name="s",
                                  num_cores=NC, num_subcores=NS)
    return pl.kernel(
        _sc_conv_body,
        out_type=jax.ShapeDtypeStruct((NC, N_PAD, F), jnp.float32),
        mesh=mesh,
        compiler_params=pltpu.CompilerParams(use_tc_tiling_on_sc=False),
        scratch_types=[
            pltpu.VMEM((CPW, CHUNK), jnp.int32),
            pltpu.VMEM((CPW, CHUNK), jnp.int32),
            pltpu.VMEM((2, CHUNK, F), jnp.float32),
            pltpu.VMEM((2, CHUNK, F), jnp.float32),
            pltpu.VMEM((2, CHUNK, F), jnp.float32),
            pltpu.VMEM((ZR, F), jnp.float32),
            pltpu.VMEM_SHARED((N_PAD, F), jnp.float32),
            pltpu.SemaphoreType.DMA,
            pltpu.SemaphoreType.DMA,
            pltpu.SemaphoreType.DMA,
            pltpu.SemaphoreType.DMA,
            pltpu.SemaphoreType.DMA,
            pltpu.SemaphoreType.DMA,
            pltpu.SemaphoreType.DMA,
        ],
    )


def _sc_conv(v, wf, s2, r2):
    return _sc_conv_kernel()(v, wf, s2, r2)


def _interact_body(agg_ref, x_ref, fw_ref, fb_ref, dw_ref, db_ref, nw_ref,
                   xo_ref, vo_ref):
    agg = _unfold(agg_ref[0] + agg_ref[1])
    t = _ssp(jnp.dot(agg, fw_ref[...], preferred_element_type=jnp.float32)
             + fb_ref[...])
    t = jnp.dot(t, dw_ref[...], preferred_element_type=jnp.float32) + db_ref[...]
    xn = x_ref[...] + t
    xo_ref[...] = xn
    v = jnp.dot(xn, nw_ref[...], preferred_element_type=jnp.float32)
    vo_ref[...] = _fold(v)


def _interact(agg2, x, fw, fb, dw, db, nw):
    return pl.pallas_call(
        _interact_body,
        grid=(NBLK,),
        in_specs=[
            pl.BlockSpec((NC, 64, 128), lambda n: (0, n, 0)),
            pl.BlockSpec((128, D), lambda n: (n, 0)),
            pl.BlockSpec((F, D), lambda n: (0, 0)),
            pl.BlockSpec((1, D), lambda n: (0, 0)),
            pl.BlockSpec((D, D), lambda n: (0, 0)),
            pl.BlockSpec((1, D), lambda n: (0, 0)),
            pl.BlockSpec((D, F), lambda n: (0, 0)),
        ],
        out_specs=[
            pl.BlockSpec((128, D), lambda n: (n, 0)),
            pl.BlockSpec((64, 128), lambda n: (n, 0)),
        ],
        out_shape=[
            jax.ShapeDtypeStruct((N_PAD, D), jnp.float32),
            jax.ShapeDtypeStruct((N_PAD // 2, 128), jnp.float32),
        ],
    )(agg2, x, fw, fb, dw, db, nw)


def _ipool_body(agg_ref, x_ref, fw_ref, fb_ref, dw_ref, db_ref,
                w1_ref, b1_ref, w2_ref, b2_ref, nm_ref, gi_ref, o_ref):
    agg = _unfold(agg_ref[0] + agg_ref[1])
    t = _ssp(jnp.dot(agg, fw_ref[...], preferred_element_type=jnp.float32)
             + fb_ref[...])
    t = jnp.dot(t, dw_ref[...], preferred_element_type=jnp.float32) + db_ref[...]
    xn = x_ref[...] + t
    h = _ssp(jnp.dot(xn, w1_ref[...],
                     preferred_element_type=jnp.float32) + b1_ref[...])
    yi = jnp.sum(h * w2_ref[...], axis=1) + b2_ref[0, 0]
    yi = yi * _col(nm_ref[...].reshape(1, 128))[:, 0]
    gir = gi_ref[...].reshape(1, 128)
    g = lax.broadcasted_iota(jnp.int32, (NG, 128), 0)
    oht = (g == gir).astype(jnp.float32)
    part = jnp.dot(oht, yi[:, None],
                   preferred_element_type=jnp.float32)
    eye = (lax.broadcasted_iota(jnp.int32, (NG, NG), 0)
           == lax.broadcasted_iota(jnp.int32, (NG, NG), 1)).astype(jnp.float32)
    prow = lax.dot_general(part, eye, _TDIMS,
                           preferred_element_type=jnp.float32)

    @pl.when(pl.program_id(0) == 0)
    def _():
        o_ref[...] = jnp.zeros_like(o_ref)

    o_ref[...] += prow


def _ipool(agg2, x, fw, fb, dw, db, w1, b1, w2r, b2, nm2, gi2):
    return pl.pallas_call(
        _ipool_body,
        grid=(NBLK,),
        in_specs=[
            pl.BlockSpec((NC, 64, 128), lambda n: (0, n, 0)),
            pl.BlockSpec((128, D), lambda n: (n, 0)),
            pl.BlockSpec((F, D), lambda n: (0, 0)),
            pl.BlockSpec((1, D), lambda n: (0, 0)),
            pl.BlockSpec((D, D), lambda n: (0, 0)),
            pl.BlockSpec((1, D), lambda n: (0, 0)),
            pl.BlockSpec((D, 32), lambda n: (0, 0)),
            pl.BlockSpec((1, 32), lambda n: (0, 0)),
            pl.BlockSpec((1, 32), lambda n: (0, 0)),
            pl.BlockSpec((1, 1), lambda n: (0, 0)),
            pl.BlockSpec((1, 1, 128), lambda n: (n, 0, 0)),
            pl.BlockSpec((1, 1, 128), lambda n: (n, 0, 0)),
        ],
        out_specs=pl.BlockSpec((1, NG), lambda n: (0, 0)),
        out_shape=jax.ShapeDtypeStruct((1, NG), jnp.float32),
    )(agg2, x, fw, fb, dw, db, w1, b1, w2r, b2, nm2, gi2)


def _pool_body(x_ref, w1_ref, b1_ref, w2_ref, b2_ref, nm_ref, gi_ref, o_ref):
    h = _ssp(jnp.dot(x_ref[...], w1_ref[...],
                     preferred_element_type=jnp.float32) + b1_ref[...])
    yi = jnp.sum(h * w2_ref[...], axis=1) + b2_ref[0, 0]
    yi = yi * _col(nm_ref[...].reshape(1, 128))[:, 0]
    gir = gi_ref[...].reshape(1, 128)
    g = lax.broadcasted_iota(jnp.int32, (NG, 128), 0)
    oht = (g == gir).astype(jnp.float32)
    part = jnp.dot(oht, yi[:, None],
                   preferred_element_type=jnp.float32)
    eye = (lax.broadcasted_iota(jnp.int32, (NG, NG), 0)
           == lax.broadcasted_iota(jnp.int32, (NG, NG), 1)).astype(jnp.float32)
    prow = lax.dot_general(part, eye, _TDIMS,
                           preferred_element_type=jnp.float32)

    @pl.when(pl.program_id(0) == 0)
    def _():
        o_ref[...] = jnp.zeros_like(o_ref)

    o_ref[...] += prow


def _pool(x, w1, b1, w2r, b2, nm2, gi2):
    return pl.pallas_call(
        _pool_body,
        grid=(NBLK,),
        in_specs=[
            pl.BlockSpec((128, D), lambda n: (n, 0)),
            pl.BlockSpec((D, 32), lambda n: (0, 0)),
            pl.BlockSpec((1, 32), lambda n: (0, 0)),
            pl.BlockSpec((1, 32), lambda n: (0, 0)),
            pl.BlockSpec((1, 1), lambda n: (0, 0)),
            pl.BlockSpec((1, 1, 128), lambda n: (n, 0, 0)),
            pl.BlockSpec((1, 1, 128), lambda n: (n, 0, 0)),
        ],
        out_specs=pl.BlockSpec((1, NG), lambda n: (0, 0)),
        out_shape=jax.ShapeDtypeStruct((1, NG), jnp.float32),
    )(x, w1, b1, w2r, b2, nm2, gi2)


def kernel(z, dR, senders, receivers, graph_idx, node_mask, edge_mask,
           embed, in2f_W, fnet_W1, fnet_b1, fnet_W2, fnet_b2,
           f2out_W, f2out_b, dense_W, dense_b,
           aw_W1, aw_b1, aw_W2, aw_b2):
    f32 = jnp.float32
    z2 = jnp.pad(z.astype(jnp.int32), (0, N_PAD - N)).reshape(NBLK, 1, 128)
    ar = jnp.arange(E_PAD, dtype=jnp.int32)
    q = (ar % 128) * (E_PAD // 128) + ar // 128
    dr2 = jnp.pad(dR, (0, E_PAD - E), constant_values=2.0 * RC)[q] \
             .reshape(EBLK, 1, EB)
    em2 = jnp.pad(edge_mask, (0, E_PAD - E))[q].reshape(EBLK, 1, EB)
    nm2 = jnp.pad(node_mask, (0, N_PAD - N)).reshape(NBLK, 1, 128)
    gi2 = jnp.pad(graph_idx.astype(jnp.int32), (0, N_PAD - N)) \
             .reshape(NBLK, 1, 128)

    emb_pad = jnp.pad(embed, ((0, 128 - MAXZ), (0, 0)))
    w1p = [jnp.pad(fnet_W1[i], ((0, GP - G), (0, 0))) for i in range(NI)]

    rb = ar % EB
    perm = q[(ar // EB) * EB + (rb % 2) * (EB // 2) + rb // 2]
    s2 = jnp.pad(senders.astype(jnp.int32), (0, E_PAD - E))[perm] \
            .reshape(NW * CPW, CHUNK)
    r2 = jnp.pad(receivers.astype(jnp.int32), (0, E_PAD - E))[perm] \
            .reshape(NW * CPW, CHUNK)

    x, v128 = _node_init(z2, emb_pad, in2f_W[0])
    wfs = [_filter1(dr2, em2, w1p[i], fnet_b1[i].reshape(1, F),
                    fnet_W2[i], fnet_b2[i].reshape(1, F)) for i in range(NI)]
    for i in range(NI - 1):
        agg = _sc_conv(v128.reshape(N_PAD, F), wfs[i].reshape(E_PAD, F),
                       s2, r2)
        x, v128 = _interact(agg.reshape(NC, N_PAD // 2, 128), x,
                            f2out_W[i], f2out_b[i].reshape(1, D),
                            dense_W[i], dense_b[i].reshape(1, D),
                            in2f_W[i + 1])
    agg = _sc_conv(v128.reshape(N_PAD, F), wfs[NI - 1].reshape(E_PAD, F),
                   s2, r2)
    out = _ipool(agg.reshape(NC, N_PAD // 2, 128), x,
                 f2out_W[NI - 1], f2out_b[NI - 1].reshape(1, D),
                 dense_W[NI - 1], dense_b[NI - 1].reshape(1, D),
                 aw_W1, aw_b1.reshape(1, 32), aw_W2.reshape(1, 32),
                 aw_b2.reshape(1, 1), nm2, gi2)
    return out.reshape(NG)
